# Initial kernel scaffold; baseline (speedup 1.0000x reference)
import jax
import jax.numpy as jnp
from jax.experimental import pallas as pl


def kernel(x, edge_index, W_l1, b_l1, W_r1, b_r1, att1, bias1, gamma, beta, prelu_a, W_l2, b_l2, W_r2, b_r2, att2, bias2):
    raise NotImplementedError("write your pallas kernel here")


# speedup vs baseline: 17.6775x; 17.6775x over previous
import functools

import jax
import jax.numpy as jnp
from jax import lax
from jax.experimental import pallas as pl
from jax.experimental.pallas import tpu as pltpu
from jax.experimental.pallas import tpu_sc as plsc

N = 10000
DIM_IN = 128
DIM_H = 16
HEADS = 10
HD = HEADS * DIM_H
DIM_OUT = 64

NP_ = 10240
NC = 2
NS = 16
NW = NC * NS
C = 128
ROWS_PER_TILE = NP_ // NS


def _ceil_div(a, b):
    return (a + b - 1) // b


def _proj_body(x_ref, wl_ref, bl_ref, wr_ref, br_ref, xl_ref, xr_ref):
    xv = x_ref[...]
    xl_ref[...] = jnp.dot(xv, wl_ref[...], preferred_element_type=jnp.float32) + bl_ref[...]
    xr_ref[...] = jnp.dot(xv, wr_ref[...], preferred_element_type=jnp.float32) + br_ref[...]


def _project(x_pad, W_l, b_l, W_r, b_r):
    n, k = x_pad.shape
    m = W_l.shape[1]
    blk = 1024
    return pl.pallas_call(
        _proj_body,
        grid=(n // blk,),
        in_specs=[
            pl.BlockSpec((blk, k), lambda i: (i, 0)),
            pl.BlockSpec((k, m), lambda i: (0, 0)),
            pl.BlockSpec((1, m), lambda i: (0, 0)),
            pl.BlockSpec((k, m), lambda i: (0, 0)),
            pl.BlockSpec((1, m), lambda i: (0, 0)),
        ],
        out_specs=[
            pl.BlockSpec((blk, m), lambda i: (i, 0)),
            pl.BlockSpec((blk, m), lambda i: (i, 0)),
        ],
        out_shape=[
            jax.ShapeDtypeStruct((n, m), jnp.float32),
            jax.ShapeDtypeStruct((n, m), jnp.float32),
        ],
    )(x_pad, W_l, b_l.reshape(1, m), W_r, b_r.reshape(1, m))


def _l1_score(XL, XR, att, src3, dst3, nch):
    mesh = plsc.VectorSubcoreMesh(core_axis_name="c", subcore_axis_name="s")

    @functools.partial(
        pl.kernel,
        out_type=[
            jax.ShapeDtypeStruct((NW, nch, C, 16), jnp.float32),
            jax.ShapeDtypeStruct((NW, 16), jnp.float32),
        ],
        mesh=mesh,
        compiler_params=pltpu.CompilerParams(needs_layout_passes=False, use_tc_tiling_on_sc=False),
        scratch_types=[
            pltpu.VMEM((C,), jnp.int32),
            pltpu.VMEM((C,), jnp.int32),
            pltpu.VMEM((C, HD), jnp.float32),
            pltpu.VMEM((C, HD), jnp.float32),
            pltpu.VMEM((C, 16), jnp.float32),
            pltpu.VMEM((HEADS, 16), jnp.float32),
            pltpu.VMEM((16,), jnp.float32),
            pltpu.SemaphoreType.DMA,
            pltpu.SemaphoreType.DMA,
        ],
    )
    def k(xl_hbm, xr_hbm, att_hbm, src_hbm, dst_hbm, e_hbm, tmax_hbm,
          src_v, dst_v, xl_v, xr_v, e_v, att_v, tm_v, sem1, sem2):
        cid = lax.axis_index("c")
        sid = lax.axis_index("s")
        wid = sid * NC + cid
        pltpu.sync_copy(att_hbm, att_v)
        zv = jnp.zeros((16,), jnp.float32)
        lanes = lax.iota(jnp.int32, 16)

        def chunk(j, gmv):
            pltpu.sync_copy(src_hbm.at[wid, j], src_v)
            pltpu.sync_copy(dst_hbm.at[wid, j], dst_v)
            cp1 = pltpu.async_copy(xl_hbm.at[src_v], xl_v, sem1)
            cp2 = pltpu.async_copy(xr_hbm.at[dst_v], xr_v, sem2)
            cp1.wait()
            cp2.wait()

            def edge(i, gmv):
                erow = zv
                for h in range(HEADS):
                    a = xl_v[i, pl.ds(h * 16, 16)]
                    b = xr_v[i, pl.ds(h * 16, 16)]
                    m = a + b
                    m = jnp.maximum(m, 0.2 * m)
                    s = jnp.sum(m * att_v[h])
                    erow = jnp.where(lanes == h, s, erow)
                e_v[i, pl.ds(0, 16)] = erow
                return jnp.maximum(gmv, erow)

            gmv = lax.fori_loop(0, C, edge, gmv)
            pltpu.sync_copy(e_v, e_hbm.at[wid, j])
            return gmv

        gmv = lax.fori_loop(0, nch, chunk, jnp.full((16,), -1e30, jnp.float32))
        tm_v[...] = gmv
        pltpu.sync_copy(tm_v, tmax_hbm.at[wid])

    return k(XL, XR, att, src3, dst3)


def _l1_agg(XLH, src4, dst4, e4, tmax1, nch2):
    mesh = plsc.VectorSubcoreMesh(core_axis_name="c", subcore_axis_name="s")
    HH = HD // 2
    WD = HH + 16

    @functools.partial(
        pl.kernel,
        out_type=jax.ShapeDtypeStruct((NC, NP_, WD), jnp.float32),
        mesh=mesh,
        compiler_params=pltpu.CompilerParams(needs_layout_passes=False, use_tc_tiling_on_sc=False),
        scratch_types=[
            pltpu.VMEM((C,), jnp.int32),
            pltpu.VMEM((C,), jnp.int32),
            pltpu.VMEM((C, HH), jnp.float32),
            pltpu.VMEM((C, 16), jnp.float32),
            pltpu.VMEM((C, WD), jnp.float32),
            pltpu.VMEM((NW, 16), jnp.float32),
            pltpu.VMEM_SHARED((NP_, WD), jnp.float32),
            pltpu.SemaphoreType.DMA,
        ],
    )
    def k(xl_hbm, src_hbm, dst_hbm, e_hbm, tmax_hbm, acc_hbm,
          src_v, dst_v, xl_v, e_v, w_v, tmax_v, acc_sh, sem1):
        cid = lax.axis_index("c")
        sid = lax.axis_index("s")
        zv = jnp.zeros((16,), jnp.float32)

        def zrow(i, carry):
            for h in range(WD // 16):
                w_v[i, pl.ds(h * 16, 16)] = zv
            return carry

        lax.fori_loop(0, C, zrow, 0)
        for t in range(ROWS_PER_TILE // C):
            pltpu.sync_copy(w_v, acc_sh.at[pl.ds(sid * ROWS_PER_TILE + t * C, C)])
        plsc.subcore_barrier()

        pltpu.sync_copy(tmax_hbm, tmax_v)
        g = tmax_v[0]
        for w in range(1, NW):
            g = jnp.maximum(g, tmax_v[w])
        lanes = lax.iota(jnp.int32, 16)
        msk = lanes < HEADS
        is_c0 = jnp.broadcast_to(cid == 0, (16,))

        def chunk(j, carry):
            pltpu.sync_copy(src_hbm.at[cid, sid, j], src_v)
            pltpu.sync_copy(dst_hbm.at[sid, j], dst_v)
            cp1 = pltpu.async_copy(xl_hbm.at[src_v], xl_v, sem1)
            pltpu.sync_copy(e_hbm.at[sid, j], e_v)
            cp1.wait()

            def edge(i, carry):
                ev = e_v[i, pl.ds(0, 16)]
                ex = jnp.where(msk, jnp.exp(ev - g), 0.0)
                w_v[i, pl.ds(HH, 16)] = ex
                for h in range(HEADS // 2):
                    sel = jnp.where(is_c0, ex[h], ex[h + 5])
                    w_v[i, pl.ds(h * 16, 16)] = xl_v[i, pl.ds(h * 16, 16)] * sel
                return carry

            lax.fori_loop(0, C, edge, 0)
            pltpu.sync_copy(w_v, acc_sh.at[dst_v], add=True)
            return carry

        lax.fori_loop(0, nch2, chunk, 0)
        plsc.subcore_barrier()
        for t in range(ROWS_PER_TILE // C):
            base = sid * ROWS_PER_TILE + t * C
            pltpu.sync_copy(acc_sh.at[pl.ds(base, C)], acc_hbm.at[cid].at[pl.ds(base, C)])

    return k(XLH, src4, dst4, e4, tmax1)


_MBLK = 1024


def _mid1_body(acc_ref, b1_ref, h_ref, s_ref):
    i = pl.program_id(0)
    cols = []
    for h in range(HEADS):
        acc = acc_ref[h // 5]
        nmr = acc[:, (h % 5) * 16:(h % 5 + 1) * 16]
        den = acc_ref[0][:, 80 + h:80 + h + 1]
        cols.append(nmr / (den + 1e-16))
    hc = jnp.concatenate(cols, axis=1) + b1_ref[...]
    h_ref[...] = hc
    ridx = lax.broadcasted_iota(jnp.int32, (_MBLK, 1), 0) + i * _MBLK
    m = (ridx < N).astype(jnp.float32)
    hm = hc * m
    ps = jnp.concatenate(
        [jnp.sum(hm, axis=0, keepdims=True),
         jnp.sum(hm * hc, axis=0, keepdims=True)], axis=0)

    @pl.when(i == 0)
    def _():
        s_ref[...] = jnp.zeros_like(s_ref)

    s_ref[...] += ps


def _mid2_body(h_ref, s_ref, g_ref, be_ref, a_ref, wl_ref, bl_ref,
               wr_ref, br_ref, xl2_ref, xr2_ref):
    mean = s_ref[0:1, :] / N
    var = s_ref[1:2, :] / N - mean * mean
    hn = (h_ref[...] - mean) / jnp.sqrt(var + 1e-5) * g_ref[...] + be_ref[...]
    a = a_ref[0, 0]
    hp = jnp.where(hn > 0, hn, a * hn)
    xl2_ref[...] = jnp.dot(hp, wl_ref[...], preferred_element_type=jnp.float32) + bl_ref[...]
    xr2_ref[...] = jnp.dot(hp, wr_ref[...], preferred_element_type=jnp.float32) + br_ref[...]


def _mid(acc1, bias1, gamma, beta, prelu_a, W_l2, b_l2, W_r2, b_r2):
    WD = 96
    g = NP_ // _MBLK
    h1, sums = pl.pallas_call(
        _mid1_body,
        grid=(g,),
        in_specs=[
            pl.BlockSpec((NC, _MBLK, WD), lambda i: (0, i, 0)),
            pl.BlockSpec((1, HD), lambda i: (0, 0)),
        ],
        out_specs=[
            pl.BlockSpec((_MBLK, HD), lambda i: (i, 0)),
            pl.BlockSpec((2, HD), lambda i: (0, 0)),
        ],
        out_shape=[
            jax.ShapeDtypeStruct((NP_, HD), jnp.float32),
            jax.ShapeDtypeStruct((2, HD), jnp.float32),
        ],
    )(acc1, bias1.reshape(1, HD))
    return pl.pallas_call(
        _mid2_body,
        grid=(g,),
        in_specs=[
            pl.BlockSpec((_MBLK, HD), lambda i: (i, 0)),
            pl.BlockSpec((2, HD), lambda i: (0, 0)),
            pl.BlockSpec((1, HD), lambda i: (0, 0)),
            pl.BlockSpec((1, HD), lambda i: (0, 0)),
            pl.BlockSpec((1, 1), lambda i: (0, 0)),
            pl.BlockSpec((HD, DIM_OUT), lambda i: (0, 0)),
            pl.BlockSpec((1, DIM_OUT), lambda i: (0, 0)),
            pl.BlockSpec((HD, DIM_OUT), lambda i: (0, 0)),
            pl.BlockSpec((1, DIM_OUT), lambda i: (0, 0)),
        ],
        out_specs=[
            pl.BlockSpec((_MBLK, DIM_OUT), lambda i: (i, 0)),
            pl.BlockSpec((_MBLK, DIM_OUT), lambda i: (i, 0)),
        ],
        out_shape=[
            jax.ShapeDtypeStruct((NP_, DIM_OUT), jnp.float32),
            jax.ShapeDtypeStruct((NP_, DIM_OUT), jnp.float32),
        ],
    )(h1, sums, gamma.reshape(1, HD), beta.reshape(1, HD),
      prelu_a.reshape(1, 1), W_l2, b_l2.reshape(1, DIM_OUT), W_r2,
      b_r2.reshape(1, DIM_OUT))


def _l2_score(XL2, XR2, att2, src3, dst3, nch):
    mesh = plsc.VectorSubcoreMesh(core_axis_name="c", subcore_axis_name="s")

    @functools.partial(
        pl.kernel,
        out_type=[
            jax.ShapeDtypeStruct((NW, nch, C), jnp.float32),
            jax.ShapeDtypeStruct((NW, 16), jnp.float32),
        ],
        mesh=mesh,
        compiler_params=pltpu.CompilerParams(needs_layout_passes=False, use_tc_tiling_on_sc=False),
        scratch_types=[
            pltpu.VMEM((C,), jnp.int32),
            pltpu.VMEM((C,), jnp.int32),
            pltpu.VMEM((C, DIM_OUT), jnp.float32),
            pltpu.VMEM((C, DIM_OUT), jnp.float32),
            pltpu.VMEM((C,), jnp.float32),
            pltpu.VMEM((4, 16), jnp.float32),
            pltpu.VMEM((16,), jnp.float32),
            pltpu.SemaphoreType.DMA,
            pltpu.SemaphoreType.DMA,
        ],
    )
    def k(xl_hbm, xr_hbm, att_hbm, src_hbm, dst_hbm, e_hbm, tmax_hbm,
          src_v, dst_v, xl_v, xr_v, e_v, att_v, tm_v, sem1, sem2):
        cid = lax.axis_index("c")
        sid = lax.axis_index("s")
        wid = sid * NC + cid
        pltpu.sync_copy(att_hbm, att_v)
        zv = jnp.zeros((16,), jnp.float32)
        lanes = lax.iota(jnp.int32, 16)

        def chunk(j, gmv):
            pltpu.sync_copy(src_hbm.at[wid, j], src_v)
            pltpu.sync_copy(dst_hbm.at[wid, j], dst_v)
            cp1 = pltpu.async_copy(xl_hbm.at[src_v], xl_v, sem1)
            cp2 = pltpu.async_copy(xr_hbm.at[dst_v], xr_v, sem2)
            cp1.wait()
            cp2.wait()

            def blk(kk, gmv):
                evec = zv
                for ii in range(16):
                    i = kk * 16 + ii
                    sv = zv
                    for h in range(4):
                        a = xl_v[i, pl.ds(h * 16, 16)]
                        b = xr_v[i, pl.ds(h * 16, 16)]
                        m = a + b
                        m = jnp.maximum(m, 0.2 * m)
                        sv = sv + m * att_v[h]
                    evec = jnp.where(lanes == ii, jnp.sum(sv), evec)
                e_v[pl.ds(kk * 16, 16)] = evec
                return jnp.maximum(gmv, evec)

            gmv = lax.fori_loop(0, C // 16, blk, gmv)
            pltpu.sync_copy(e_v, e_hbm.at[wid, j])
            return gmv

        gmv = lax.fori_loop(0, nch, chunk, jnp.full((16,), -1e30, jnp.float32))
        tm_v[...] = gmv
        pltpu.sync_copy(tm_v, tmax_hbm.at[wid])

    return k(XL2, XR2, att2.reshape(4, 16), src3, dst3)


def _l2_agg(XL2, src3, dst3, e2, tmax2, nch):
    mesh = plsc.VectorSubcoreMesh(core_axis_name="c", subcore_axis_name="s")
    WD = DIM_OUT + 16

    @functools.partial(
        pl.kernel,
        out_type=jax.ShapeDtypeStruct((NC, NP_, WD), jnp.float32),
        mesh=mesh,
        compiler_params=pltpu.CompilerParams(needs_layout_passes=False, use_tc_tiling_on_sc=False),
        scratch_types=[
            pltpu.VMEM((C,), jnp.int32),
            pltpu.VMEM((C,), jnp.int32),
            pltpu.VMEM((C, DIM_OUT), jnp.float32),
            pltpu.VMEM((C // 16, 16), jnp.float32),
            pltpu.VMEM((C,), jnp.float32),
            pltpu.VMEM((C, WD), jnp.float32),
            pltpu.VMEM((NW, 16), jnp.float32),
            pltpu.VMEM_SHARED((NP_, WD), jnp.float32),
            pltpu.SemaphoreType.DMA,
        ],
    )
    def k(xl_hbm, src_hbm, dst_hbm, e_hbm, tmax_hbm, acc_hbm,
          src_v, dst_v, xl_v, e_v, ex_v, w_v, tmax_v, acc_sh, sem1):
        del ex_v
        cid = lax.axis_index("c")
        sid = lax.axis_index("s")
        wid = sid * NC + cid
        zv = jnp.zeros((16,), jnp.float32)

        def zrow(i, carry):
            for h in range(WD // 16):
                w_v[i, pl.ds(h * 16, 16)] = zv
            return carry

        lax.fori_loop(0, C, zrow, 0)
        for t in range(ROWS_PER_TILE // C):
            pltpu.sync_copy(w_v, acc_sh.at[pl.ds(sid * ROWS_PER_TILE + t * C, C)])
        plsc.subcore_barrier()

        pltpu.sync_copy(tmax_hbm, tmax_v)
        g = tmax_v[0]
        for w in range(1, NW):
            g = jnp.maximum(g, tmax_v[w])
        g = jnp.max(g)
        lanes = lax.iota(jnp.int32, 16)
        mask0 = jnp.where(lanes == 0, 1.0, 0.0)

        def chunk(j, carry):
            pltpu.sync_copy(src_hbm.at[wid, j], src_v)
            pltpu.sync_copy(dst_hbm.at[wid, j], dst_v)
            cp1 = pltpu.async_copy(xl_hbm.at[src_v], xl_v, sem1)
            pltpu.sync_copy(e_hbm.at[wid, j], e_v)
            cp1.wait()

            def blk(kk, carry):
                ev = e_v[kk, pl.ds(0, 16)]
                exv = jnp.exp(ev - g)
                for ii in range(16):
                    i = kk * 16 + ii
                    sc = exv[ii]
                    w_v[i, pl.ds(DIM_OUT, 16)] = mask0 * sc
                    for h in range(4):
                        w_v[i, pl.ds(h * 16, 16)] = xl_v[i, pl.ds(h * 16, 16)] * sc
                return carry

            lax.fori_loop(0, C // 16, blk, 0)
            pltpu.sync_copy(w_v, acc_sh.at[dst_v], add=True)
            return carry

        lax.fori_loop(0, nch, chunk, 0)
        plsc.subcore_barrier()
        for t in range(ROWS_PER_TILE // C):
            base = sid * ROWS_PER_TILE + t * C
            pltpu.sync_copy(acc_sh.at[pl.ds(base, C)], acc_hbm.at[cid].at[pl.ds(base, C)])

    return k(XL2, src3, dst3, e2.reshape(NW, nch, C // 16, 16), tmax2)


def _post_body(acc_ref, b2_ref, out_ref, logp_ref):
    acc = acc_ref[0] + acc_ref[1]
    o = acc[:N, :DIM_OUT] / (acc[:N, DIM_OUT:DIM_OUT + 1] + 1e-16) + b2_ref[...]
    out_ref[...] = o
    mx = jnp.max(o, axis=1, keepdims=True)
    om = o - mx
    logp_ref[...] = om - jnp.log(jnp.sum(jnp.exp(om), axis=1, keepdims=True))


def _post(acc2, bias2):
    return pl.pallas_call(
        _post_body,
        out_shape=[
            jax.ShapeDtypeStruct((N, DIM_OUT), jnp.float32),
            jax.ShapeDtypeStruct((N, DIM_OUT), jnp.float32),
        ],
    )(acc2, bias2.reshape(1, DIM_OUT))


def kernel(x, edge_index, W_l1, b_l1, W_r1, b_r1, att1, bias1, gamma, beta,
           prelu_a, W_l2, b_l2, W_r2, b_r2, att2, bias2):
    n = x.shape[0]
    e_tot = edge_index.shape[1] + n
    nch = _ceil_div(e_tot, NW * C)
    e_pad = NW * nch * C
    loop = jnp.arange(n, dtype=jnp.int32)
    pad = jnp.full((e_pad - e_tot,), N, jnp.int32)
    src3 = jnp.concatenate([edge_index[0].astype(jnp.int32), loop, pad]).reshape(NW, nch, C)
    dst3 = jnp.concatenate([edge_index[1].astype(jnp.int32), loop, pad]).reshape(NW, nch, C)
    x_pad = jnp.pad(x, ((0, NP_ - n), (0, 0)))

    nch2 = e_pad // (NS * C)
    src_flat = src3.reshape(e_pad)
    src4 = jnp.stack([src_flat, src_flat + NP_]).reshape(NC, NS, nch2, C)
    dst4 = dst3.reshape(NS, nch2, C)

    XL1, XR1 = _project(x_pad, W_l1, b_l1, W_r1, b_r1)
    e1, tmax1 = _l1_score(XL1, XR1, att1, src3, dst3, nch)
    XLH = jnp.concatenate([XL1[:, :80], XL1[:, 80:]], axis=0)
    e4 = e1.reshape(NS, nch2, C, 16)
    acc1 = _l1_agg(XLH, src4, dst4, e4, tmax1, nch2)
    XL2, XR2 = _mid(acc1, bias1, gamma, beta, prelu_a, W_l2, b_l2, W_r2, b_r2)
    e2, tmax2 = _l2_score(XL2, XR2, att2, src3, dst3, nch)
    acc2 = _l2_agg(XL2, src3, dst3, e2, tmax2, nch)
    return _post(acc2, bias2)


# speedup vs baseline: 24.9205x; 1.4097x over previous
import functools

import jax
import jax.numpy as jnp
from jax import lax
from jax.experimental import pallas as pl
from jax.experimental.pallas import tpu as pltpu
from jax.experimental.pallas import tpu_sc as plsc

N = 10000
DIM_IN = 128
DIM_H = 16
HEADS = 10
HD = HEADS * DIM_H
DIM_OUT = 64

NP_ = 10240
NC = 2
NS = 16
NW = NC * NS
C = 128
ROWS_PER_TILE = NP_ // NS


def _ceil_div(a, b):
    return (a + b - 1) // b


def _proj_body(x_ref, wl_ref, bl_ref, wr_ref, br_ref, xl_ref, xr_ref):
    xv = x_ref[...]
    xl_ref[...] = jnp.dot(xv, wl_ref[...], preferred_element_type=jnp.float32) + bl_ref[...]
    xr_ref[...] = jnp.dot(xv, wr_ref[...], preferred_element_type=jnp.float32) + br_ref[...]


def _project(x_pad, W_l, b_l, W_r, b_r):
    n, k = x_pad.shape
    m = W_l.shape[1]
    blk = 1024
    return pl.pallas_call(
        _proj_body,
        grid=(n // blk,),
        in_specs=[
            pl.BlockSpec((blk, k), lambda i: (i, 0)),
            pl.BlockSpec((k, m), lambda i: (0, 0)),
            pl.BlockSpec((1, m), lambda i: (0, 0)),
            pl.BlockSpec((k, m), lambda i: (0, 0)),
            pl.BlockSpec((1, m), lambda i: (0, 0)),
        ],
        out_specs=[
            pl.BlockSpec((blk, m), lambda i: (i, 0)),
            pl.BlockSpec((blk, m), lambda i: (i, 0)),
        ],
        out_shape=[
            jax.ShapeDtypeStruct((n, m), jnp.float32),
            jax.ShapeDtypeStruct((n, m), jnp.float32),
        ],
    )(x_pad, W_l, b_l.reshape(1, m), W_r, b_r.reshape(1, m))


def _l1_score(XL, XR, att, src3, dst3, nch):
    mesh = plsc.VectorSubcoreMesh(core_axis_name="c", subcore_axis_name="s")

    @functools.partial(
        pl.kernel,
        out_type=[
            jax.ShapeDtypeStruct((NW, nch, C, 16), jnp.float32),
            jax.ShapeDtypeStruct((NW, 16), jnp.float32),
        ],
        mesh=mesh,
        compiler_params=pltpu.CompilerParams(needs_layout_passes=False, use_tc_tiling_on_sc=False),
        scratch_types=[
            pltpu.VMEM((nch, C), jnp.int32),
            pltpu.VMEM((nch, C), jnp.int32),
            pltpu.VMEM((C, HD), jnp.float32),
            pltpu.VMEM((C, HD), jnp.float32),
            pltpu.VMEM((C, HD), jnp.float32),
            pltpu.VMEM((C, HD), jnp.float32),
            pltpu.VMEM((C, 16), jnp.float32),
            pltpu.VMEM((C, 16), jnp.float32),
            pltpu.VMEM((HEADS, 16), jnp.float32),
            pltpu.VMEM((16,), jnp.float32),
            pltpu.SemaphoreType.DMA,
            pltpu.SemaphoreType.DMA,
            pltpu.SemaphoreType.DMA,
            pltpu.SemaphoreType.DMA,
        ],
    )
    def k(xl_hbm, xr_hbm, att_hbm, src_hbm, dst_hbm, e_hbm, tmax_hbm,
          isrc, idst, xl0, xl1, xr0, xr1, e0, e1, att_v, tm_v,
          sl0, sl1, sr0, sr1):
        cid = lax.axis_index("c")
        sid = lax.axis_index("s")
        wid = sid * NC + cid
        pltpu.sync_copy(att_hbm, att_v)
        pltpu.sync_copy(src_hbm.at[wid], isrc)
        pltpu.sync_copy(dst_hbm.at[wid], idst)
        zv = jnp.zeros((16,), jnp.float32)
        lanes = lax.iota(jnp.int32, 16)
        bufs = ((xl0, xr0, e0, sl0, sr0), (xl1, xr1, e1, sl1, sr1))

        def issue(j, b):
            xl_v, xr_v, _, sl, sr = bufs[b]
            pltpu.async_copy(xl_hbm.at[isrc.at[j]], xl_v, sl)
            pltpu.async_copy(xr_hbm.at[idst.at[j]], xr_v, sr)

        issue(0, 0)
        issue(1, 1)

        def step(j, b, gmv):
            xl_v, xr_v, e_v, sl, sr = bufs[b]
            pltpu.make_async_copy(xl_hbm.at[isrc.at[j]], xl_v, sl).wait()
            pltpu.make_async_copy(xr_hbm.at[idst.at[j]], xr_v, sr).wait()

            def edge(i, gmv):
                erow = zv
                for h in range(HEADS):
                    a = xl_v[i, pl.ds(h * 16, 16)]
                    b_ = xr_v[i, pl.ds(h * 16, 16)]
                    m = a + b_
                    m = jnp.maximum(m, 0.2 * m)
                    s = jnp.sum(m * att_v[h])
                    erow = jnp.where(lanes == h, s, erow)
                e_v[i, pl.ds(0, 16)] = erow
                return jnp.maximum(gmv, erow)

            gmv = lax.fori_loop(0, C, edge, gmv)
            pltpu.sync_copy(e_v, e_hbm.at[wid, j])

            @pl.when(j + 2 < nch)
            def _():
                issue(j + 2, b)

            return gmv

        def body2(jj, gmv):
            gmv = step(2 * jj, 0, gmv)
            return step(2 * jj + 1, 1, gmv)

        gmv = lax.fori_loop(0, nch // 2, body2,
                            jnp.full((16,), -1e30, jnp.float32))
        tm_v[...] = gmv
        pltpu.sync_copy(tm_v, tmax_hbm.at[wid])

    return k(XL, XR, att, src3, dst3)


def _l1_agg(XLH, src4, dst4, e4, tmax1, nch2):
    mesh = plsc.VectorSubcoreMesh(core_axis_name="c", subcore_axis_name="s")
    HH = HD // 2
    WD = HH + 16

    @functools.partial(
        pl.kernel,
        out_type=jax.ShapeDtypeStruct((NC, NP_, WD), jnp.float32),
        mesh=mesh,
        compiler_params=pltpu.CompilerParams(needs_layout_passes=False, use_tc_tiling_on_sc=False),
        scratch_types=[
            pltpu.VMEM((C,), jnp.int32),
            pltpu.VMEM((C,), jnp.int32),
            pltpu.VMEM((C,), jnp.int32),
            pltpu.VMEM((C,), jnp.int32),
            pltpu.VMEM((C, HH), jnp.float32),
            pltpu.VMEM((C, HH), jnp.float32),
            pltpu.VMEM((C, 16), jnp.float32),
            pltpu.VMEM((C, 16), jnp.float32),
            pltpu.VMEM((C, WD), jnp.float32),
            pltpu.VMEM((NW, 16), jnp.float32),
            pltpu.VMEM_SHARED((NP_, WD), jnp.float32),
            pltpu.SemaphoreType.DMA,
            pltpu.SemaphoreType.DMA,
            pltpu.SemaphoreType.DMA,
            pltpu.SemaphoreType.DMA,
        ],
    )
    def k(xl_hbm, src_hbm, dst_hbm, e_hbm, tmax_hbm, acc_hbm,
          src0, src1, dst0, dst1, xl0, xl1, e0, e1, w_v, tmax_v, acc_sh,
          sl0, sl1, se0, se1):
        cid = lax.axis_index("c")
        sid = lax.axis_index("s")
        zv = jnp.zeros((16,), jnp.float32)

        def zrow(i, carry):
            for h in range(WD // 16):
                w_v[i, pl.ds(h * 16, 16)] = zv
            return carry

        lax.fori_loop(0, C, zrow, 0)
        for t in range(ROWS_PER_TILE // C):
            pltpu.sync_copy(w_v, acc_sh.at[pl.ds(sid * ROWS_PER_TILE + t * C, C)])
        plsc.subcore_barrier()

        pltpu.sync_copy(tmax_hbm, tmax_v)
        g = tmax_v[0]
        for w in range(1, NW):
            g = jnp.maximum(g, tmax_v[w])
        lanes = lax.iota(jnp.int32, 16)
        msk = lanes < HEADS
        is_c0 = jnp.broadcast_to(cid == 0, (16,))
        bufs = ((src0, dst0, xl0, e0, sl0, se0), (src1, dst1, xl1, e1, sl1, se1))

        def issue(j, b):
            src_v, dst_v, xl_v2, e_v, sl, se = bufs[b]
            pltpu.sync_copy(src_hbm.at[cid, sid, j], src_v)
            pltpu.sync_copy(dst_hbm.at[sid, j], dst_v)
            pltpu.async_copy(xl_hbm.at[src_v], xl_v2, sl)
            pltpu.async_copy(e_hbm.at[sid, j], e_v, se)

        issue(0, 0)
        issue(1, 1)

        def step(j, b):
            src_v, dst_v, xl_v2, e_v, sl, se = bufs[b]
            pltpu.make_async_copy(xl_hbm.at[src_v], xl_v2, sl).wait()
            pltpu.make_async_copy(e_hbm.at[sid, j], e_v, se).wait()

            def edge(i, carry):
                ev = e_v[i, pl.ds(0, 16)]
                ex = jnp.where(msk, jnp.exp(ev - g), 0.0)
                w_v[i, pl.ds(HH, 16)] = ex
                for h in range(HEADS // 2):
                    sel = jnp.where(is_c0, ex[h], ex[h + 5])
                    w_v[i, pl.ds(h * 16, 16)] = xl_v2[i, pl.ds(h * 16, 16)] * sel
                return carry

            lax.fori_loop(0, C, edge, 0)
            pltpu.sync_copy(w_v, acc_sh.at[dst_v], add=True)

            @pl.when(j + 2 < nch2)
            def _():
                issue(j + 2, b)

        def body2(jj, carry):
            step(2 * jj, 0)
            step(2 * jj + 1, 1)
            return carry

        lax.fori_loop(0, nch2 // 2, body2, 0)
        plsc.subcore_barrier()
        for t in range(ROWS_PER_TILE // C):
            base = sid * ROWS_PER_TILE + t * C
            pltpu.sync_copy(acc_sh.at[pl.ds(base, C)], acc_hbm.at[cid].at[pl.ds(base, C)])

    return k(XLH, src4, dst4, e4, tmax1)


_MBLK = 1024


def _mid1_body(acc_ref, b1_ref, h_ref, s_ref):
    i = pl.program_id(0)
    cols = []
    for h in range(HEADS):
        acc = acc_ref[h // 5]
        nmr = acc[:, (h % 5) * 16:(h % 5 + 1) * 16]
        den = acc_ref[0][:, 80 + h:80 + h + 1]
        cols.append(nmr / (den + 1e-16))
    hc = jnp.concatenate(cols, axis=1) + b1_ref[...]
    h_ref[...] = hc
    ridx = lax.broadcasted_iota(jnp.int32, (_MBLK, 1), 0) + i * _MBLK
    m = (ridx < N).astype(jnp.float32)
    hm = hc * m
    ps = jnp.concatenate(
        [jnp.sum(hm, axis=0, keepdims=True),
         jnp.sum(hm * hc, axis=0, keepdims=True)], axis=0)

    @pl.when(i == 0)
    def _():
        s_ref[...] = jnp.zeros_like(s_ref)

    s_ref[...] += ps


def _mid2_body(h_ref, s_ref, g_ref, be_ref, a_ref, wl_ref, bl_ref,
               wr_ref, br_ref, xl2_ref, xr2_ref):
    mean = s_ref[0:1, :] / N
    var = s_ref[1:2, :] / N - mean * mean
    hn = (h_ref[...] - mean) / jnp.sqrt(var + 1e-5) * g_ref[...] + be_ref[...]
    a = a_ref[0, 0]
    hp = jnp.where(hn > 0, hn, a * hn)
    xl2_ref[...] = jnp.dot(hp, wl_ref[...], preferred_element_type=jnp.float32) + bl_ref[...]
    xr2_ref[...] = jnp.dot(hp, wr_ref[...], preferred_element_type=jnp.float32) + br_ref[...]


def _mid(acc1, bias1, gamma, beta, prelu_a, W_l2, b_l2, W_r2, b_r2):
    WD = 96
    g = NP_ // _MBLK
    h1, sums = pl.pallas_call(
        _mid1_body,
        grid=(g,),
        in_specs=[
            pl.BlockSpec((NC, _MBLK, WD), lambda i: (0, i, 0)),
            pl.BlockSpec((1, HD), lambda i: (0, 0)),
        ],
        out_specs=[
            pl.BlockSpec((_MBLK, HD), lambda i: (i, 0)),
            pl.BlockSpec((2, HD), lambda i: (0, 0)),
        ],
        out_shape=[
            jax.ShapeDtypeStruct((NP_, HD), jnp.float32),
            jax.ShapeDtypeStruct((2, HD), jnp.float32),
        ],
    )(acc1, bias1.reshape(1, HD))
    return pl.pallas_call(
        _mid2_body,
        grid=(g,),
        in_specs=[
            pl.BlockSpec((_MBLK, HD), lambda i: (i, 0)),
            pl.BlockSpec((2, HD), lambda i: (0, 0)),
            pl.BlockSpec((1, HD), lambda i: (0, 0)),
            pl.BlockSpec((1, HD), lambda i: (0, 0)),
            pl.BlockSpec((1, 1), lambda i: (0, 0)),
            pl.BlockSpec((HD, DIM_OUT), lambda i: (0, 0)),
            pl.BlockSpec((1, DIM_OUT), lambda i: (0, 0)),
            pl.BlockSpec((HD, DIM_OUT), lambda i: (0, 0)),
            pl.BlockSpec((1, DIM_OUT), lambda i: (0, 0)),
        ],
        out_specs=[
            pl.BlockSpec((_MBLK, DIM_OUT), lambda i: (i, 0)),
            pl.BlockSpec((_MBLK, DIM_OUT), lambda i: (i, 0)),
        ],
        out_shape=[
            jax.ShapeDtypeStruct((NP_, DIM_OUT), jnp.float32),
            jax.ShapeDtypeStruct((NP_, DIM_OUT), jnp.float32),
        ],
    )(h1, sums, gamma.reshape(1, HD), beta.reshape(1, HD),
      prelu_a.reshape(1, 1), W_l2, b_l2.reshape(1, DIM_OUT), W_r2,
      b_r2.reshape(1, DIM_OUT))


def _l2_score(XL2, XR2, att2, src3, dst3, nch):
    mesh = plsc.VectorSubcoreMesh(core_axis_name="c", subcore_axis_name="s")

    @functools.partial(
        pl.kernel,
        out_type=[
            jax.ShapeDtypeStruct((NW, nch, C), jnp.float32),
            jax.ShapeDtypeStruct((NW, 16), jnp.float32),
        ],
        mesh=mesh,
        compiler_params=pltpu.CompilerParams(needs_layout_passes=False, use_tc_tiling_on_sc=False),
        scratch_types=[
            pltpu.VMEM((nch, C), jnp.int32),
            pltpu.VMEM((nch, C), jnp.int32),
            pltpu.VMEM((C, DIM_OUT), jnp.float32),
            pltpu.VMEM((C, DIM_OUT), jnp.float32),
            pltpu.VMEM((C, DIM_OUT), jnp.float32),
            pltpu.VMEM((C, DIM_OUT), jnp.float32),
            pltpu.VMEM((C,), jnp.float32),
            pltpu.VMEM((C,), jnp.float32),
            pltpu.VMEM((4, 16), jnp.float32),
            pltpu.VMEM((16,), jnp.float32),
            pltpu.SemaphoreType.DMA,
            pltpu.SemaphoreType.DMA,
            pltpu.SemaphoreType.DMA,
            pltpu.SemaphoreType.DMA,
        ],
    )
    def k(xl_hbm, xr_hbm, att_hbm, src_hbm, dst_hbm, e_hbm, tmax_hbm,
          isrc, idst, xl0, xl1, xr0, xr1, e0, e1, att_v, tm_v,
          sl0, sl1, sr0, sr1):
        cid = lax.axis_index("c")
        sid = lax.axis_index("s")
        wid = sid * NC + cid
        pltpu.sync_copy(att_hbm, att_v)
        pltpu.sync_copy(src_hbm.at[wid], isrc)
        pltpu.sync_copy(dst_hbm.at[wid], idst)
        zv = jnp.zeros((16,), jnp.float32)
        lanes = lax.iota(jnp.int32, 16)
        bufs = ((xl0, xr0, e0, sl0, sr0), (xl1, xr1, e1, sl1, sr1))

        def issue(j, b):
            xl_v, xr_v, _, sl, sr = bufs[b]
            pltpu.async_copy(xl_hbm.at[isrc.at[j]], xl_v, sl)
            pltpu.async_copy(xr_hbm.at[idst.at[j]], xr_v, sr)

        issue(0, 0)
        issue(1, 1)

        def step(j, b, gmv):
            xl_v, xr_v, e_v, sl, sr = bufs[b]
            pltpu.make_async_copy(xl_hbm.at[isrc.at[j]], xl_v, sl).wait()
            pltpu.make_async_copy(xr_hbm.at[idst.at[j]], xr_v, sr).wait()

            def blk(kk, gmv):
                evec = zv
                for ii in range(16):
                    i = kk * 16 + ii
                    sv = zv
                    for h in range(4):
                        a = xl_v[i, pl.ds(h * 16, 16)]
                        b_ = xr_v[i, pl.ds(h * 16, 16)]
                        m = a + b_
                        m = jnp.maximum(m, 0.2 * m)
                        sv = sv + m * att_v[h]
                    evec = jnp.where(lanes == ii, jnp.sum(sv), evec)
                e_v[pl.ds(kk * 16, 16)] = evec
                return jnp.maximum(gmv, evec)

            gmv = lax.fori_loop(0, C // 16, blk, gmv)
            pltpu.sync_copy(e_v, e_hbm.at[wid, j])

            @pl.when(j + 2 < nch)
            def _():
                issue(j + 2, b)

            return gmv

        def body2(jj, gmv):
            gmv = step(2 * jj, 0, gmv)
            return step(2 * jj + 1, 1, gmv)

        gmv = lax.fori_loop(0, nch // 2, body2,
                            jnp.full((16,), -1e30, jnp.float32))
        tm_v[...] = gmv
        pltpu.sync_copy(tm_v, tmax_hbm.at[wid])

    return k(XL2, XR2, att2.reshape(4, 16), src3, dst3)


def _l2_agg(XL2, src3, dst3, e2, tmax2, nch):
    mesh = plsc.VectorSubcoreMesh(core_axis_name="c", subcore_axis_name="s")
    WD = DIM_OUT + 16

    @functools.partial(
        pl.kernel,
        out_type=jax.ShapeDtypeStruct((NC, NP_, WD), jnp.float32),
        mesh=mesh,
        compiler_params=pltpu.CompilerParams(needs_layout_passes=False, use_tc_tiling_on_sc=False),
        scratch_types=[
            pltpu.VMEM((nch, C), jnp.int32),
            pltpu.VMEM((nch, C), jnp.int32),
            pltpu.VMEM((C, DIM_OUT), jnp.float32),
            pltpu.VMEM((C, DIM_OUT), jnp.float32),
            pltpu.VMEM((C // 16, 16), jnp.float32),
            pltpu.VMEM((C // 16, 16), jnp.float32),
            pltpu.VMEM((C, WD), jnp.float32),
            pltpu.VMEM((NW, 16), jnp.float32),
            pltpu.VMEM_SHARED((NP_, WD), jnp.float32),
            pltpu.SemaphoreType.DMA,
            pltpu.SemaphoreType.DMA,
            pltpu.SemaphoreType.DMA,
            pltpu.SemaphoreType.DMA,
        ],
    )
    def k(xl_hbm, src_hbm, dst_hbm, e_hbm, tmax_hbm, acc_hbm,
          isrc, idst, xl0, xl1, e0, e1, w_v, tmax_v, acc_sh,
          sl0, sl1, se0, se1):
        cid = lax.axis_index("c")
        sid = lax.axis_index("s")
        wid = sid * NC + cid
        zv = jnp.zeros((16,), jnp.float32)

        def zrow(i, carry):
            for h in range(WD // 16):
                w_v[i, pl.ds(h * 16, 16)] = zv
            return carry

        lax.fori_loop(0, C, zrow, 0)
        for t in range(ROWS_PER_TILE // C):
            pltpu.sync_copy(w_v, acc_sh.at[pl.ds(sid * ROWS_PER_TILE + t * C, C)])
        plsc.subcore_barrier()

        pltpu.sync_copy(src_hbm.at[wid], isrc)
        pltpu.sync_copy(dst_hbm.at[wid], idst)
        pltpu.sync_copy(tmax_hbm, tmax_v)
        g = tmax_v[0]
        for w in range(1, NW):
            g = jnp.maximum(g, tmax_v[w])
        g = jnp.max(g)
        lanes = lax.iota(jnp.int32, 16)
        mask0 = jnp.where(lanes == 0, 1.0, 0.0)
        bufs = ((xl0, e0, sl0, se0), (xl1, e1, sl1, se1))

        def issue(j, b):
            xl_v, e_v, sl, se = bufs[b]
            pltpu.async_copy(xl_hbm.at[isrc.at[j]], xl_v, sl)
            pltpu.async_copy(e_hbm.at[wid, j], e_v, se)

        issue(0, 0)
        issue(1, 1)

        def step(j, b):
            xl_v, e_v, sl, se = bufs[b]
            pltpu.make_async_copy(xl_hbm.at[isrc.at[j]], xl_v, sl).wait()
            pltpu.make_async_copy(e_hbm.at[wid, j], e_v, se).wait()

            def blk(kk, carry):
                ev = e_v[kk, pl.ds(0, 16)]
                exv = jnp.exp(ev - g)
                for ii in range(16):
                    i = kk * 16 + ii
                    sc = exv[ii]
                    w_v[i, pl.ds(DIM_OUT, 16)] = mask0 * sc
                    for h in range(4):
                        w_v[i, pl.ds(h * 16, 16)] = xl_v[i, pl.ds(h * 16, 16)] * sc
                return carry

            lax.fori_loop(0, C // 16, blk, 0)
            pltpu.sync_copy(w_v, acc_sh.at[idst.at[j]], add=True)

            @pl.when(j + 2 < nch)
            def _():
                issue(j + 2, b)

        def body2(jj, carry):
            step(2 * jj, 0)
            step(2 * jj + 1, 1)
            return carry

        lax.fori_loop(0, nch // 2, body2, 0)
        plsc.subcore_barrier()
        for t in range(ROWS_PER_TILE // C):
            base = sid * ROWS_PER_TILE + t * C
            pltpu.sync_copy(acc_sh.at[pl.ds(base, C)], acc_hbm.at[cid].at[pl.ds(base, C)])

    return k(XL2, src3, dst3, e2.reshape(NW, nch, C // 16, 16), tmax2)


def _post_body(acc_ref, b2_ref, out_ref, logp_ref):
    acc = acc_ref[0] + acc_ref[1]
    o = acc[:N, :DIM_OUT] / (acc[:N, DIM_OUT:DIM_OUT + 1] + 1e-16) + b2_ref[...]
    out_ref[...] = o
    mx = jnp.max(o, axis=1, keepdims=True)
    om = o - mx
    logp_ref[...] = om - jnp.log(jnp.sum(jnp.exp(om), axis=1, keepdims=True))


def _post(acc2, bias2):
    return pl.pallas_call(
        _post_body,
        out_shape=[
            jax.ShapeDtypeStruct((N, DIM_OUT), jnp.float32),
            jax.ShapeDtypeStruct((N, DIM_OUT), jnp.float32),
        ],
    )(acc2, bias2.reshape(1, DIM_OUT))


def kernel(x, edge_index, W_l1, b_l1, W_r1, b_r1, att1, bias1, gamma, beta,
           prelu_a, W_l2, b_l2, W_r2, b_r2, att2, bias2):
    n = x.shape[0]
    e_tot = edge_index.shape[1] + n
    nch = _ceil_div(e_tot, NW * C)
    e_pad = NW * nch * C
    loop = jnp.arange(n, dtype=jnp.int32)
    pad = jnp.full((e_pad - e_tot,), N, jnp.int32)
    src3 = jnp.concatenate([edge_index[0].astype(jnp.int32), loop, pad]).reshape(NW, nch, C)
    dst3 = jnp.concatenate([edge_index[1].astype(jnp.int32), loop, pad]).reshape(NW, nch, C)
    x_pad = jnp.pad(x, ((0, NP_ - n), (0, 0)))

    nch2 = e_pad // (NS * C)
    src_flat = src3.reshape(e_pad)
    src4 = jnp.stack([src_flat, src_flat + NP_]).reshape(NC, NS, nch2, C)
    dst4 = dst3.reshape(NS, nch2, C)

    XL1, XR1 = _project(x_pad, W_l1, b_l1, W_r1, b_r1)
    e1, tmax1 = _l1_score(XL1, XR1, att1, src3, dst3, nch)
    XLH = jnp.concatenate([XL1[:, :80], XL1[:, 80:]], axis=0)
    e4 = e1.reshape(NS, nch2, C, 16)
    acc1 = _l1_agg(XLH, src4, dst4, e4, tmax1, nch2)
    XL2, XR2 = _mid(acc1, bias1, gamma, beta, prelu_a, W_l2, b_l2, W_r2, b_r2)
    e2, tmax2 = _l2_score(XL2, XR2, att2, src3, dst3, nch)
    acc2 = _l2_agg(XL2, src3, dst3, e2, tmax2, nch)
    return _post(acc2, bias2)


# speedup vs baseline: 35.6882x; 1.4321x over previous
import functools

import jax
import jax.numpy as jnp
from jax import lax
from jax.experimental import pallas as pl
from jax.experimental.pallas import tpu as pltpu
from jax.experimental.pallas import tpu_sc as plsc

N = 10000
DIM_IN = 128
DIM_H = 16
HEADS = 10
HD = HEADS * DIM_H
DIM_OUT = 64

NP_ = 10240
NC = 2
NS = 16
NW = NC * NS
C = 128
ROWS_PER_TILE = NP_ // NS


def _ceil_div(a, b):
    return (a + b - 1) // b


def _proj_body(x_ref, wl_ref, bl_ref, wr_ref, br_ref, xl_ref, xr_ref):
    xv = x_ref[...]
    xl_ref[...] = jnp.dot(xv, wl_ref[...], preferred_element_type=jnp.float32) + bl_ref[...]
    xr_ref[...] = jnp.dot(xv, wr_ref[...], preferred_element_type=jnp.float32) + br_ref[...]


def _project(x_pad, W_l, b_l, W_r, b_r):
    n, k = x_pad.shape
    m = W_l.shape[1]
    blk = 1024
    return pl.pallas_call(
        _proj_body,
        grid=(n // blk,),
        in_specs=[
            pl.BlockSpec((blk, k), lambda i: (i, 0)),
            pl.BlockSpec((k, m), lambda i: (0, 0)),
            pl.BlockSpec((1, m), lambda i: (0, 0)),
            pl.BlockSpec((k, m), lambda i: (0, 0)),
            pl.BlockSpec((1, m), lambda i: (0, 0)),
        ],
        out_specs=[
            pl.BlockSpec((blk, m), lambda i: (i, 0)),
            pl.BlockSpec((blk, m), lambda i: (i, 0)),
        ],
        out_shape=[
            jax.ShapeDtypeStruct((n, m), jnp.float32),
            jax.ShapeDtypeStruct((n, m), jnp.float32),
        ],
    )(x_pad, W_l, b_l.reshape(1, m), W_r, b_r.reshape(1, m))


def _l1_score(XL, XR, att, src3, dst3, nch):
    mesh = plsc.VectorSubcoreMesh(core_axis_name="c", subcore_axis_name="s")

    @functools.partial(
        pl.kernel,
        out_type=[
            jax.ShapeDtypeStruct((NW, nch, C, 16), jnp.float32),
            jax.ShapeDtypeStruct((NW, 16), jnp.float32),
        ],
        mesh=mesh,
        compiler_params=pltpu.CompilerParams(needs_layout_passes=False, use_tc_tiling_on_sc=False),
        scratch_types=[
            pltpu.VMEM((nch, C), jnp.int32),
            pltpu.VMEM((nch, C), jnp.int32),
            pltpu.VMEM((C, HD), jnp.float32),
            pltpu.VMEM((C, HD), jnp.float32),
            pltpu.VMEM((C, HD), jnp.float32),
            pltpu.VMEM((C, HD), jnp.float32),
            pltpu.VMEM((C, 16), jnp.float32),
            pltpu.VMEM((C, 16), jnp.float32),
            pltpu.VMEM((HEADS, 16), jnp.float32),
            pltpu.VMEM((16,), jnp.float32),
            pltpu.SemaphoreType.DMA,
            pltpu.SemaphoreType.DMA,
            pltpu.SemaphoreType.DMA,
            pltpu.SemaphoreType.DMA,
        ],
    )
    def k(xl_hbm, xr_hbm, att_hbm, src_hbm, dst_hbm, e_hbm, tmax_hbm,
          isrc, idst, xl0, xl1, xr0, xr1, e0, e1, att_v, tm_v,
          sl0, sl1, sr0, sr1):
        cid = lax.axis_index("c")
        sid = lax.axis_index("s")
        wid = sid * NC + cid
        pltpu.sync_copy(att_hbm, att_v)
        pltpu.sync_copy(src_hbm.at[wid], isrc)
        pltpu.sync_copy(dst_hbm.at[wid], idst)
        zv = jnp.zeros((16,), jnp.float32)
        lanes = lax.iota(jnp.int32, 16)
        atts = tuple(att_v[h] for h in range(HEADS))
        bufs = ((xl0, xr0, e0, sl0, sr0), (xl1, xr1, e1, sl1, sr1))

        def issue(j, b):
            xl_v, xr_v, _, sl, sr = bufs[b]
            pltpu.async_copy(xl_hbm.at[isrc.at[j]], xl_v, sl)
            pltpu.async_copy(xr_hbm.at[idst.at[j]], xr_v, sr)

        issue(0, 0)
        issue(1, 1)

        def step(j, b, gmv):
            xl_v, xr_v, e_v, sl, sr = bufs[b]
            pltpu.make_async_copy(xl_hbm.at[isrc.at[j]], xl_v, sl).wait()
            pltpu.make_async_copy(xr_hbm.at[idst.at[j]], xr_v, sr).wait()

            @plsc.parallel_loop(0, C, 1, unroll=2, carry=gmv)
            def gmv(i, gm):
                erow = zv
                for h in range(HEADS):
                    a = xl_v[i, pl.ds(h * 16, 16)]
                    b_ = xr_v[i, pl.ds(h * 16, 16)]
                    m = a + b_
                    m = jnp.maximum(m, 0.2 * m)
                    s = jnp.sum(m * atts[h])
                    erow = jnp.where(lanes == h, s, erow)
                e_v[i, pl.ds(0, 16)] = erow
                return jnp.maximum(gm, erow)

            pltpu.sync_copy(e_v, e_hbm.at[wid, j])

            @pl.when(j + 2 < nch)
            def _():
                issue(j + 2, b)

            return gmv

        def body2(jj, gmv):
            gmv = step(2 * jj, 0, gmv)
            return step(2 * jj + 1, 1, gmv)

        gmv = lax.fori_loop(0, nch // 2, body2,
                            jnp.full((16,), -1e30, jnp.float32))
        tm_v[...] = gmv
        pltpu.sync_copy(tm_v, tmax_hbm.at[wid])

    return k(XL, XR, att, src3, dst3)


def _l1_agg(XLH, src4, dst4, e4, tmax1, nch2):
    mesh = plsc.VectorSubcoreMesh(core_axis_name="c", subcore_axis_name="s")
    HH = HD // 2
    WD = HH + 16

    @functools.partial(
        pl.kernel,
        out_type=jax.ShapeDtypeStruct((NC, NP_, WD), jnp.float32),
        mesh=mesh,
        compiler_params=pltpu.CompilerParams(needs_layout_passes=False, use_tc_tiling_on_sc=False),
        scratch_types=[
            pltpu.VMEM((C,), jnp.int32),
            pltpu.VMEM((C,), jnp.int32),
            pltpu.VMEM((C,), jnp.int32),
            pltpu.VMEM((C,), jnp.int32),
            pltpu.VMEM((C,), jnp.int32),
            pltpu.VMEM((C,), jnp.int32),
            pltpu.VMEM((C, HH), jnp.float32),
            pltpu.VMEM((C, HH), jnp.float32),
            pltpu.VMEM((C, 16), jnp.float32),
            pltpu.VMEM((C, 16), jnp.float32),
            pltpu.VMEM((C, WD), jnp.float32),
            pltpu.VMEM((C, WD), jnp.float32),
            pltpu.VMEM((NW, 16), jnp.float32),
            pltpu.VMEM_SHARED((NP_, WD), jnp.float32),
            pltpu.SemaphoreType.DMA,
            pltpu.SemaphoreType.DMA,
            pltpu.SemaphoreType.DMA,
            pltpu.SemaphoreType.DMA,
            pltpu.SemaphoreType.DMA,
            pltpu.SemaphoreType.DMA,
        ],
    )
    def k(xl_hbm, src_hbm, dst_hbm, e_hbm, tmax_hbm, acc_hbm,
          src0, src1, dst0, dst1, dsc0, dsc1, xl0, xl1, e0, e1, w0, w1,
          tmax_v, acc_sh, sl0, sl1, se0, se1, ss0, ss1):
        cid = lax.axis_index("c")
        sid = lax.axis_index("s")
        zv = jnp.zeros((16,), jnp.float32)

        def zrow(i, carry):
            for h in range(WD // 16):
                w0[i, pl.ds(h * 16, 16)] = zv
            return carry

        lax.fori_loop(0, C, zrow, 0)
        for t in range(ROWS_PER_TILE // C):
            pltpu.sync_copy(w0, acc_sh.at[pl.ds(sid * ROWS_PER_TILE + t * C, C)])
        plsc.subcore_barrier()

        pltpu.sync_copy(tmax_hbm, tmax_v)
        g = tmax_v[0]
        for w in range(1, NW):
            g = jnp.maximum(g, tmax_v[w])
        lanes = lax.iota(jnp.int32, 16)
        msk = lanes < HEADS
        is_c0 = jnp.broadcast_to(cid == 0, (16,))
        bufs = ((src0, dst0, dsc0, xl0, e0, w0, sl0, se0, ss0),
                (src1, dst1, dsc1, xl1, e1, w1, sl1, se1, ss1))

        def issue(j, b):
            src_v, dst_v, _, xl_v2, e_v, _, sl, se, _ = bufs[b]
            pltpu.sync_copy(src_hbm.at[cid, sid, j], src_v)
            pltpu.sync_copy(dst_hbm.at[sid, j], dst_v)
            pltpu.async_copy(xl_hbm.at[src_v], xl_v2, sl)
            pltpu.async_copy(e_hbm.at[sid, j], e_v, se)

        issue(0, 0)
        issue(1, 1)

        def step(j, b):
            src_v, dst_v, dsc, xl_v2, e_v, w_v, sl, se, ss = bufs[b]
            pltpu.make_async_copy(xl_hbm.at[src_v], xl_v2, sl).wait()
            pltpu.make_async_copy(e_hbm.at[sid, j], e_v, se).wait()

            @pl.when(j >= 2)
            def _():
                pltpu.make_async_copy(w_v, acc_sh.at[dsc], ss).wait()

            @plsc.parallel_loop(0, C, 1, unroll=2)
            def _(i):
                ev = e_v[i, pl.ds(0, 16)]
                ex = jnp.where(msk, jnp.exp(ev - g), 0.0)
                w_v[i, pl.ds(HH, 16)] = ex
                for h in range(HEADS // 2):
                    sel = jnp.where(is_c0, ex[h], ex[h + 5])
                    w_v[i, pl.ds(h * 16, 16)] = xl_v2[i, pl.ds(h * 16, 16)] * sel

            for kk in range(C // 16):
                dsc[pl.ds(kk * 16, 16)] = dst_v[pl.ds(kk * 16, 16)]
            pltpu.async_copy(w_v, acc_sh.at[dsc], ss, add=True)

            @pl.when(j + 2 < nch2)
            def _():
                issue(j + 2, b)

        def body2(jj, carry):
            step(2 * jj, 0)
            step(2 * jj + 1, 1)
            return carry

        lax.fori_loop(0, nch2 // 2, body2, 0)
        pltpu.make_async_copy(w0, acc_sh.at[dsc0], ss0).wait()
        pltpu.make_async_copy(w1, acc_sh.at[dsc1], ss1).wait()
        plsc.subcore_barrier()
        for t in range(ROWS_PER_TILE // C):
            base = sid * ROWS_PER_TILE + t * C
            pltpu.sync_copy(acc_sh.at[pl.ds(base, C)], acc_hbm.at[cid].at[pl.ds(base, C)])

    return k(XLH, src4, dst4, e4, tmax1)


_MBLK = 1024


def _mid1_body(acc_ref, b1_ref, h_ref, s_ref):
    i = pl.program_id(0)
    cols = []
    for h in range(HEADS):
        acc = acc_ref[h // 5]
        nmr = acc[:, (h % 5) * 16:(h % 5 + 1) * 16]
        den = acc_ref[0][:, 80 + h:80 + h + 1]
        cols.append(nmr / (den + 1e-16))
    hc = jnp.concatenate(cols, axis=1) + b1_ref[...]
    h_ref[...] = hc
    ridx = lax.broadcasted_iota(jnp.int32, (_MBLK, 1), 0) + i * _MBLK
    m = (ridx < N).astype(jnp.float32)
    hm = hc * m
    ps = jnp.concatenate(
        [jnp.sum(hm, axis=0, keepdims=True),
         jnp.sum(hm * hc, axis=0, keepdims=True)], axis=0)

    @pl.when(i == 0)
    def _():
        s_ref[...] = jnp.zeros_like(s_ref)

    s_ref[...] += ps


def _mid2_body(h_ref, s_ref, g_ref, be_ref, a_ref, wl_ref, bl_ref,
               wr_ref, br_ref, xl2_ref, xr2_ref):
    mean = s_ref[0:1, :] / N
    var = s_ref[1:2, :] / N - mean * mean
    hn = (h_ref[...] - mean) / jnp.sqrt(var + 1e-5) * g_ref[...] + be_ref[...]
    a = a_ref[0, 0]
    hp = jnp.where(hn > 0, hn, a * hn)
    xl2_ref[...] = jnp.dot(hp, wl_ref[...], preferred_element_type=jnp.float32) + bl_ref[...]
    xr2_ref[...] = jnp.dot(hp, wr_ref[...], preferred_element_type=jnp.float32) + br_ref[...]


def _mid(acc1, bias1, gamma, beta, prelu_a, W_l2, b_l2, W_r2, b_r2):
    WD = 96
    g = NP_ // _MBLK
    h1, sums = pl.pallas_call(
        _mid1_body,
        grid=(g,),
        in_specs=[
            pl.BlockSpec((NC, _MBLK, WD), lambda i: (0, i, 0)),
            pl.BlockSpec((1, HD), lambda i: (0, 0)),
        ],
        out_specs=[
            pl.BlockSpec((_MBLK, HD), lambda i: (i, 0)),
            pl.BlockSpec((2, HD), lambda i: (0, 0)),
        ],
        out_shape=[
            jax.ShapeDtypeStruct((NP_, HD), jnp.float32),
            jax.ShapeDtypeStruct((2, HD), jnp.float32),
        ],
    )(acc1, bias1.reshape(1, HD))
    return pl.pallas_call(
        _mid2_body,
        grid=(g,),
        in_specs=[
            pl.BlockSpec((_MBLK, HD), lambda i: (i, 0)),
            pl.BlockSpec((2, HD), lambda i: (0, 0)),
            pl.BlockSpec((1, HD), lambda i: (0, 0)),
            pl.BlockSpec((1, HD), lambda i: (0, 0)),
            pl.BlockSpec((1, 1), lambda i: (0, 0)),
            pl.BlockSpec((HD, DIM_OUT), lambda i: (0, 0)),
            pl.BlockSpec((1, DIM_OUT), lambda i: (0, 0)),
            pl.BlockSpec((HD, DIM_OUT), lambda i: (0, 0)),
            pl.BlockSpec((1, DIM_OUT), lambda i: (0, 0)),
        ],
        out_specs=[
            pl.BlockSpec((_MBLK, DIM_OUT), lambda i: (i, 0)),
            pl.BlockSpec((_MBLK, DIM_OUT), lambda i: (i, 0)),
        ],
        out_shape=[
            jax.ShapeDtypeStruct((NP_, DIM_OUT), jnp.float32),
            jax.ShapeDtypeStruct((NP_, DIM_OUT), jnp.float32),
        ],
    )(h1, sums, gamma.reshape(1, HD), beta.reshape(1, HD),
      prelu_a.reshape(1, 1), W_l2, b_l2.reshape(1, DIM_OUT), W_r2,
      b_r2.reshape(1, DIM_OUT))


def _l2_score(XL2, XR2, att2, src3, dst3, nch):
    mesh = plsc.VectorSubcoreMesh(core_axis_name="c", subcore_axis_name="s")

    @functools.partial(
        pl.kernel,
        out_type=[
            jax.ShapeDtypeStruct((NW, nch, C), jnp.float32),
            jax.ShapeDtypeStruct((NW, 16), jnp.float32),
        ],
        mesh=mesh,
        compiler_params=pltpu.CompilerParams(needs_layout_passes=False, use_tc_tiling_on_sc=False),
        scratch_types=[
            pltpu.VMEM((nch, C), jnp.int32),
            pltpu.VMEM((nch, C), jnp.int32),
            pltpu.VMEM((C, DIM_OUT), jnp.float32),
            pltpu.VMEM((C, DIM_OUT), jnp.float32),
            pltpu.VMEM((C, DIM_OUT), jnp.float32),
            pltpu.VMEM((C, DIM_OUT), jnp.float32),
            pltpu.VMEM((C,), jnp.float32),
            pltpu.VMEM((C,), jnp.float32),
            pltpu.VMEM((4, 16), jnp.float32),
            pltpu.VMEM((16,), jnp.float32),
            pltpu.SemaphoreType.DMA,
            pltpu.SemaphoreType.DMA,
            pltpu.SemaphoreType.DMA,
            pltpu.SemaphoreType.DMA,
        ],
    )
    def k(xl_hbm, xr_hbm, att_hbm, src_hbm, dst_hbm, e_hbm, tmax_hbm,
          isrc, idst, xl0, xl1, xr0, xr1, e0, e1, att_v, tm_v,
          sl0, sl1, sr0, sr1):
        cid = lax.axis_index("c")
        sid = lax.axis_index("s")
        wid = sid * NC + cid
        pltpu.sync_copy(att_hbm, att_v)
        pltpu.sync_copy(src_hbm.at[wid], isrc)
        pltpu.sync_copy(dst_hbm.at[wid], idst)
        zv = jnp.zeros((16,), jnp.float32)
        lanes = lax.iota(jnp.int32, 16)
        atts = tuple(att_v[h] for h in range(4))
        bufs = ((xl0, xr0, e0, sl0, sr0), (xl1, xr1, e1, sl1, sr1))

        def issue(j, b):
            xl_v, xr_v, _, sl, sr = bufs[b]
            pltpu.async_copy(xl_hbm.at[isrc.at[j]], xl_v, sl)
            pltpu.async_copy(xr_hbm.at[idst.at[j]], xr_v, sr)

        issue(0, 0)
        issue(1, 1)

        def step(j, b, gmv):
            xl_v, xr_v, e_v, sl, sr = bufs[b]
            pltpu.make_async_copy(xl_hbm.at[isrc.at[j]], xl_v, sl).wait()
            pltpu.make_async_copy(xr_hbm.at[idst.at[j]], xr_v, sr).wait()

            @plsc.parallel_loop(0, C // 16, 1, unroll=1, carry=gmv)
            def gmv(kk, gm):
                evec = zv
                for ii in range(16):
                    i = kk * 16 + ii
                    sv = zv
                    for h in range(4):
                        a = xl_v[i, pl.ds(h * 16, 16)]
                        b_ = xr_v[i, pl.ds(h * 16, 16)]
                        m = a + b_
                        m = jnp.maximum(m, 0.2 * m)
                        sv = sv + m * atts[h]
                    evec = jnp.where(lanes == ii, jnp.sum(sv), evec)
                e_v[pl.ds(kk * 16, 16)] = evec
                return jnp.maximum(gm, evec)

            pltpu.sync_copy(e_v, e_hbm.at[wid, j])

            @pl.when(j + 2 < nch)
            def _():
                issue(j + 2, b)

            return gmv

        def body2(jj, gmv):
            gmv = step(2 * jj, 0, gmv)
            return step(2 * jj + 1, 1, gmv)

        gmv = lax.fori_loop(0, nch // 2, body2,
                            jnp.full((16,), -1e30, jnp.float32))
        tm_v[...] = gmv
        pltpu.sync_copy(tm_v, tmax_hbm.at[wid])

    return k(XL2, XR2, att2.reshape(4, 16), src3, dst3)


def _l2_agg(XL2, src3, dst3, e2, tmax2, nch):
    mesh = plsc.VectorSubcoreMesh(core_axis_name="c", subcore_axis_name="s")
    WD = DIM_OUT + 16

    @functools.partial(
        pl.kernel,
        out_type=jax.ShapeDtypeStruct((NC, NP_, WD), jnp.float32),
        mesh=mesh,
        compiler_params=pltpu.CompilerParams(needs_layout_passes=False, use_tc_tiling_on_sc=False),
        scratch_types=[
            pltpu.VMEM((nch, C), jnp.int32),
            pltpu.VMEM((nch, C), jnp.int32),
            pltpu.VMEM((C, DIM_OUT), jnp.float32),
            pltpu.VMEM((C, DIM_OUT), jnp.float32),
            pltpu.VMEM((C // 16, 16), jnp.float32),
            pltpu.VMEM((C // 16, 16), jnp.float32),
            pltpu.VMEM((C, WD), jnp.float32),
            pltpu.VMEM((C, WD), jnp.float32),
            pltpu.VMEM((NW, 16), jnp.float32),
            pltpu.VMEM_SHARED((NP_, WD), jnp.float32),
            pltpu.SemaphoreType.DMA,
            pltpu.SemaphoreType.DMA,
            pltpu.SemaphoreType.DMA,
            pltpu.SemaphoreType.DMA,
            pltpu.SemaphoreType.DMA,
            pltpu.SemaphoreType.DMA,
        ],
    )
    def k(xl_hbm, src_hbm, dst_hbm, e_hbm, tmax_hbm, acc_hbm,
          isrc, idst, xl0, xl1, e0, e1, w0, w1, tmax_v, acc_sh,
          sl0, sl1, se0, se1, ss0, ss1):
        cid = lax.axis_index("c")
        sid = lax.axis_index("s")
        wid = sid * NC + cid
        zv = jnp.zeros((16,), jnp.float32)

        def zrow(i, carry):
            for h in range(WD // 16):
                w0[i, pl.ds(h * 16, 16)] = zv
            return carry

        lax.fori_loop(0, C, zrow, 0)
        for t in range(ROWS_PER_TILE // C):
            pltpu.sync_copy(w0, acc_sh.at[pl.ds(sid * ROWS_PER_TILE + t * C, C)])
        plsc.subcore_barrier()

        pltpu.sync_copy(src_hbm.at[wid], isrc)
        pltpu.sync_copy(dst_hbm.at[wid], idst)
        pltpu.sync_copy(tmax_hbm, tmax_v)
        g = tmax_v[0]
        for w in range(1, NW):
            g = jnp.maximum(g, tmax_v[w])
        g = jnp.max(g)
        lanes = lax.iota(jnp.int32, 16)
        mask0 = jnp.where(lanes == 0, 1.0, 0.0)
        bufs = ((xl0, e0, w0, sl0, se0, ss0), (xl1, e1, w1, sl1, se1, ss1))

        def issue(j, b):
            xl_v, e_v, _, sl, se, _ = bufs[b]
            pltpu.async_copy(xl_hbm.at[isrc.at[j]], xl_v, sl)
            pltpu.async_copy(e_hbm.at[wid, j], e_v, se)

        issue(0, 0)
        issue(1, 1)

        def step(j, b):
            xl_v, e_v, w_v, sl, se, ss = bufs[b]
            pltpu.make_async_copy(xl_hbm.at[isrc.at[j]], xl_v, sl).wait()
            pltpu.make_async_copy(e_hbm.at[wid, j], e_v, se).wait()

            @pl.when(j >= 2)
            def _():
                pltpu.make_async_copy(w_v, acc_sh.at[idst.at[j - 2]], ss).wait()

            @plsc.parallel_loop(0, C // 16, 1, unroll=2)
            def _(kk):
                ev = e_v[kk, pl.ds(0, 16)]
                exv = jnp.exp(ev - g)
                for ii in range(16):
                    i = kk * 16 + ii
                    sc = exv[ii]
                    w_v[i, pl.ds(DIM_OUT, 16)] = mask0 * sc
                    for h in range(4):
                        w_v[i, pl.ds(h * 16, 16)] = xl_v[i, pl.ds(h * 16, 16)] * sc

            pltpu.async_copy(w_v, acc_sh.at[idst.at[j]], ss, add=True)

            @pl.when(j + 2 < nch)
            def _():
                issue(j + 2, b)

        def body2(jj, carry):
            step(2 * jj, 0)
            step(2 * jj + 1, 1)
            return carry

        lax.fori_loop(0, nch // 2, body2, 0)
        pltpu.make_async_copy(w0, acc_sh.at[idst.at[nch - 2]], ss0).wait()
        pltpu.make_async_copy(w1, acc_sh.at[idst.at[nch - 1]], ss1).wait()
        plsc.subcore_barrier()
        for t in range(ROWS_PER_TILE // C):
            base = sid * ROWS_PER_TILE + t * C
            pltpu.sync_copy(acc_sh.at[pl.ds(base, C)], acc_hbm.at[cid].at[pl.ds(base, C)])

    return k(XL2, src3, dst3, e2.reshape(NW, nch, C // 16, 16), tmax2)


def _post_body(acc_ref, b2_ref, out_ref, logp_ref):
    acc = acc_ref[0] + acc_ref[1]
    o = acc[:N, :DIM_OUT] / (acc[:N, DIM_OUT:DIM_OUT + 1] + 1e-16) + b2_ref[...]
    out_ref[...] = o
    mx = jnp.max(o, axis=1, keepdims=True)
    om = o - mx
    logp_ref[...] = om - jnp.log(jnp.sum(jnp.exp(om), axis=1, keepdims=True))


def _post(acc2, bias2):
    return pl.pallas_call(
        _post_body,
        out_shape=[
            jax.ShapeDtypeStruct((N, DIM_OUT), jnp.float32),
            jax.ShapeDtypeStruct((N, DIM_OUT), jnp.float32),
        ],
    )(acc2, bias2.reshape(1, DIM_OUT))


def kernel(x, edge_index, W_l1, b_l1, W_r1, b_r1, att1, bias1, gamma, beta,
           prelu_a, W_l2, b_l2, W_r2, b_r2, att2, bias2):
    n = x.shape[0]
    e_tot = edge_index.shape[1] + n
    nch = _ceil_div(e_tot, NW * C)
    e_pad = NW * nch * C
    loop = jnp.arange(n, dtype=jnp.int32)
    pad = jnp.full((e_pad - e_tot,), N, jnp.int32)
    src3 = jnp.concatenate([edge_index[0].astype(jnp.int32), loop, pad]).reshape(NW, nch, C)
    dst3 = jnp.concatenate([edge_index[1].astype(jnp.int32), loop, pad]).reshape(NW, nch, C)
    x_pad = jnp.pad(x, ((0, NP_ - n), (0, 0)))

    nch2 = e_pad // (NS * C)
    src_flat = src3.reshape(e_pad)
    src4 = jnp.stack([src_flat, src_flat + NP_]).reshape(NC, NS, nch2, C)
    dst4 = dst3.reshape(NS, nch2, C)

    XL1, XR1 = _project(x_pad, W_l1, b_l1, W_r1, b_r1)
    e1, tmax1 = _l1_score(XL1, XR1, att1, src3, dst3, nch)
    XLH = jnp.concatenate([XL1[:, :80], XL1[:, 80:]], axis=0)
    e4 = e1.reshape(NS, nch2, C, 16)
    acc1 = _l1_agg(XLH, src4, dst4, e4, tmax1, nch2)
    XL2, XR2 = _mid(acc1, bias1, gamma, beta, prelu_a, W_l2, b_l2, W_r2, b_r2)
    e2, tmax2 = _l2_score(XL2, XR2, att2, src3, dst3, nch)
    acc2 = _l2_agg(XL2, src3, dst3, e2, tmax2, nch)
    return _post(acc2, bias2)


# speedup vs baseline: 38.1359x; 1.0686x over previous
import functools

import jax
import jax.numpy as jnp
from jax import lax
from jax.experimental import pallas as pl
from jax.experimental.pallas import tpu as pltpu
from jax.experimental.pallas import tpu_sc as plsc

N = 10000
DIM_IN = 128
DIM_H = 16
HEADS = 10
HD = HEADS * DIM_H
DIM_OUT = 64

NP_ = 10240
NC = 2
NS = 16
NW = NC * NS
C = 128
ROWS_PER_TILE = NP_ // NS


def _ceil_div(a, b):
    return (a + b - 1) // b


def _proj_body(x_ref, wl_ref, bl_ref, wr_ref, br_ref, xl_ref, xr_ref):
    xv = x_ref[...]
    xl_ref[...] = jnp.dot(xv, wl_ref[...], preferred_element_type=jnp.float32) + bl_ref[...]
    xr_ref[...] = jnp.dot(xv, wr_ref[...], preferred_element_type=jnp.float32) + br_ref[...]


def _project(x_pad, W_l, b_l, W_r, b_r):
    n, k = x_pad.shape
    m = W_l.shape[1]
    blk = 1024
    return pl.pallas_call(
        _proj_body,
        grid=(n // blk,),
        in_specs=[
            pl.BlockSpec((blk, k), lambda i: (i, 0)),
            pl.BlockSpec((k, m), lambda i: (0, 0)),
            pl.BlockSpec((1, m), lambda i: (0, 0)),
            pl.BlockSpec((k, m), lambda i: (0, 0)),
            pl.BlockSpec((1, m), lambda i: (0, 0)),
        ],
        out_specs=[
            pl.BlockSpec((blk, m), lambda i: (i, 0)),
            pl.BlockSpec((blk, m), lambda i: (i, 0)),
        ],
        out_shape=[
            jax.ShapeDtypeStruct((n, m), jnp.float32),
            jax.ShapeDtypeStruct((n, m), jnp.float32),
        ],
    )(x_pad, W_l, b_l.reshape(1, m), W_r, b_r.reshape(1, m))


def _l1_score(XL, XR, att, src3, dst3, nch):
    mesh = plsc.VectorSubcoreMesh(core_axis_name="c", subcore_axis_name="s")

    @functools.partial(
        pl.kernel,
        out_type=[
            jax.ShapeDtypeStruct((NW, nch, C, 16), jnp.float32),
            jax.ShapeDtypeStruct((NW, 16), jnp.float32),
        ],
        mesh=mesh,
        compiler_params=pltpu.CompilerParams(needs_layout_passes=False, use_tc_tiling_on_sc=False),
        scratch_types=[
            pltpu.VMEM((nch, C), jnp.int32),
            pltpu.VMEM((nch, C), jnp.int32),
            pltpu.VMEM((C, HD), jnp.float32),
            pltpu.VMEM((C, HD), jnp.float32),
            pltpu.VMEM((C, HD), jnp.float32),
            pltpu.VMEM((C, HD), jnp.float32),
            pltpu.VMEM((C, 16), jnp.float32),
            pltpu.VMEM((C, 16), jnp.float32),
            pltpu.VMEM((HEADS, 16), jnp.float32),
            pltpu.VMEM((16,), jnp.float32),
            pltpu.SemaphoreType.DMA,
            pltpu.SemaphoreType.DMA,
            pltpu.SemaphoreType.DMA,
            pltpu.SemaphoreType.DMA,
        ],
    )
    def k(xl_hbm, xr_hbm, att_hbm, src_hbm, dst_hbm, e_hbm, tmax_hbm,
          isrc, idst, xl0, xl1, xr0, xr1, e0, e1, att_v, tm_v,
          sl0, sl1, sr0, sr1):
        cid = lax.axis_index("c")
        sid = lax.axis_index("s")
        wid = sid * NC + cid
        pltpu.sync_copy(att_hbm, att_v)
        pltpu.sync_copy(src_hbm.at[wid], isrc)
        pltpu.sync_copy(dst_hbm.at[wid], idst)
        zv = jnp.zeros((16,), jnp.float32)
        lanes = lax.iota(jnp.int32, 16)
        atts = tuple(att_v[h] for h in range(HEADS))
        bufs = ((xl0, xr0, e0, sl0, sr0), (xl1, xr1, e1, sl1, sr1))

        def issue(j, b):
            xl_v, xr_v, _, sl, sr = bufs[b]
            pltpu.async_copy(xl_hbm.at[isrc.at[j]], xl_v, sl)
            pltpu.async_copy(xr_hbm.at[idst.at[j]], xr_v, sr)

        issue(0, 0)
        issue(1, 1)

        def step(j, b, gmv):
            xl_v, xr_v, e_v, sl, sr = bufs[b]
            pltpu.make_async_copy(xl_hbm.at[isrc.at[j]], xl_v, sl).wait()
            pltpu.make_async_copy(xr_hbm.at[idst.at[j]], xr_v, sr).wait()

            @plsc.parallel_loop(0, C, 1, unroll=4, carry=gmv)
            def gmv(i, gm):
                erow = zv
                for h in range(HEADS):
                    a = xl_v[i, pl.ds(h * 16, 16)]
                    b_ = xr_v[i, pl.ds(h * 16, 16)]
                    m = a + b_
                    m = jnp.maximum(m, 0.2 * m)
                    s = jnp.sum(m * atts[h])
                    erow = jnp.where(lanes == h, s, erow)
                e_v[i, pl.ds(0, 16)] = erow
                return jnp.maximum(gm, erow)

            pltpu.sync_copy(e_v, e_hbm.at[wid, j])

            @pl.when(j + 2 < nch)
            def _():
                issue(j + 2, b)

            return gmv

        def body2(jj, gmv):
            gmv = step(2 * jj, 0, gmv)
            return step(2 * jj + 1, 1, gmv)

        gmv = lax.fori_loop(0, nch // 2, body2,
                            jnp.full((16,), -1e30, jnp.float32))
        tm_v[...] = gmv
        pltpu.sync_copy(tm_v, tmax_hbm.at[wid])

    return k(XL, XR, att, src3, dst3)


def _l1_agg(XLH, src4, dst4, e4, tmax1, nch2):
    mesh = plsc.VectorSubcoreMesh(core_axis_name="c", subcore_axis_name="s")
    HH = HD // 2
    WD = HH + 16

    @functools.partial(
        pl.kernel,
        out_type=jax.ShapeDtypeStruct((NC, NP_, WD), jnp.float32),
        mesh=mesh,
        compiler_params=pltpu.CompilerParams(needs_layout_passes=False, use_tc_tiling_on_sc=False),
        scratch_types=[
            pltpu.VMEM((C,), jnp.int32),
            pltpu.VMEM((C,), jnp.int32),
            pltpu.VMEM((C,), jnp.int32),
            pltpu.VMEM((C,), jnp.int32),
            pltpu.VMEM((C,), jnp.int32),
            pltpu.VMEM((C,), jnp.int32),
            pltpu.VMEM((C, HH), jnp.float32),
            pltpu.VMEM((C, HH), jnp.float32),
            pltpu.VMEM((C, 16), jnp.float32),
            pltpu.VMEM((C, 16), jnp.float32),
            pltpu.VMEM((C, WD), jnp.float32),
            pltpu.VMEM((C, WD), jnp.float32),
            pltpu.VMEM((NW, 16), jnp.float32),
            pltpu.VMEM_SHARED((NP_, WD), jnp.float32),
            pltpu.SemaphoreType.DMA,
            pltpu.SemaphoreType.DMA,
            pltpu.SemaphoreType.DMA,
            pltpu.SemaphoreType.DMA,
            pltpu.SemaphoreType.DMA,
            pltpu.SemaphoreType.DMA,
        ],
    )
    def k(xl_hbm, src_hbm, dst_hbm, e_hbm, tmax_hbm, acc_hbm,
          src0, src1, dst0, dst1, dsc0, dsc1, xl0, xl1, e0, e1, w0, w1,
          tmax_v, acc_sh, sl0, sl1, se0, se1, ss0, ss1):
        cid = lax.axis_index("c")
        sid = lax.axis_index("s")
        zv = jnp.zeros((16,), jnp.float32)

        def zrow(i, carry):
            for h in range(WD // 16):
                w0[i, pl.ds(h * 16, 16)] = zv
            return carry

        lax.fori_loop(0, C, zrow, 0)
        for t in range(ROWS_PER_TILE // C):
            pltpu.sync_copy(w0, acc_sh.at[pl.ds(sid * ROWS_PER_TILE + t * C, C)])
        plsc.subcore_barrier()

        pltpu.sync_copy(tmax_hbm, tmax_v)
        g = tmax_v[0]
        for w in range(1, NW):
            g = jnp.maximum(g, tmax_v[w])
        lanes = lax.iota(jnp.int32, 16)
        msk = lanes < HEADS
        is_c0 = jnp.broadcast_to(cid == 0, (16,))
        bufs = ((src0, dst0, dsc0, xl0, e0, w0, sl0, se0, ss0),
                (src1, dst1, dsc1, xl1, e1, w1, sl1, se1, ss1))

        def issue(j, b):
            src_v, dst_v, _, xl_v2, e_v, _, sl, se, _ = bufs[b]
            pltpu.sync_copy(src_hbm.at[cid, sid, j], src_v)
            pltpu.sync_copy(dst_hbm.at[sid, j], dst_v)
            pltpu.async_copy(xl_hbm.at[src_v], xl_v2, sl)
            pltpu.async_copy(e_hbm.at[sid, j], e_v, se)

        issue(0, 0)
        issue(1, 1)

        def step(j, b):
            src_v, dst_v, dsc, xl_v2, e_v, w_v, sl, se, ss = bufs[b]
            pltpu.make_async_copy(xl_hbm.at[src_v], xl_v2, sl).wait()
            pltpu.make_async_copy(e_hbm.at[sid, j], e_v, se).wait()

            @pl.when(j >= 2)
            def _():
                pltpu.make_async_copy(w_v, acc_sh.at[dsc], ss).wait()

            @plsc.parallel_loop(0, C, 1, unroll=4)
            def _(i):
                ev = e_v[i, pl.ds(0, 16)]
                ex = jnp.where(msk, jnp.exp(ev - g), 0.0)
                w_v[i, pl.ds(HH, 16)] = ex
                for h in range(HEADS // 2):
                    sel = jnp.where(is_c0, ex[h], ex[h + 5])
                    w_v[i, pl.ds(h * 16, 16)] = xl_v2[i, pl.ds(h * 16, 16)] * sel

            for kk in range(C // 16):
                dsc[pl.ds(kk * 16, 16)] = dst_v[pl.ds(kk * 16, 16)]
            pltpu.async_copy(w_v, acc_sh.at[dsc], ss, add=True)

            @pl.when(j + 2 < nch2)
            def _():
                issue(j + 2, b)

        def body2(jj, carry):
            step(2 * jj, 0)
            step(2 * jj + 1, 1)
            return carry

        lax.fori_loop(0, nch2 // 2, body2, 0)
        pltpu.make_async_copy(w0, acc_sh.at[dsc0], ss0).wait()
        pltpu.make_async_copy(w1, acc_sh.at[dsc1], ss1).wait()
        plsc.subcore_barrier()
        for t in range(ROWS_PER_TILE // C):
            base = sid * ROWS_PER_TILE + t * C
            pltpu.sync_copy(acc_sh.at[pl.ds(base, C)], acc_hbm.at[cid].at[pl.ds(base, C)])

    return k(XLH, src4, dst4, e4, tmax1)


_MBLK = 1024


def _mid1_body(acc_ref, b1_ref, h_ref, s_ref):
    i = pl.program_id(0)
    cols = []
    for h in range(HEADS):
        acc = acc_ref[h // 5]
        nmr = acc[:, (h % 5) * 16:(h % 5 + 1) * 16]
        den = acc_ref[0][:, 80 + h:80 + h + 1]
        cols.append(nmr / (den + 1e-16))
    hc = jnp.concatenate(cols, axis=1) + b1_ref[...]
    h_ref[...] = hc
    ridx = lax.broadcasted_iota(jnp.int32, (_MBLK, 1), 0) + i * _MBLK
    m = (ridx < N).astype(jnp.float32)
    hm = hc * m
    ps = jnp.concatenate(
        [jnp.sum(hm, axis=0, keepdims=True),
         jnp.sum(hm * hc, axis=0, keepdims=True)], axis=0)

    @pl.when(i == 0)
    def _():
        s_ref[...] = jnp.zeros_like(s_ref)

    s_ref[...] += ps


def _mid2_body(h_ref, s_ref, g_ref, be_ref, a_ref, wl_ref, bl_ref,
               wr_ref, br_ref, xl2_ref, xr2_ref):
    mean = s_ref[0:1, :] / N
    var = s_ref[1:2, :] / N - mean * mean
    hn = (h_ref[...] - mean) / jnp.sqrt(var + 1e-5) * g_ref[...] + be_ref[...]
    a = a_ref[0, 0]
    hp = jnp.where(hn > 0, hn, a * hn)
    xl2_ref[...] = jnp.dot(hp, wl_ref[...], preferred_element_type=jnp.float32) + bl_ref[...]
    xr2_ref[...] = jnp.dot(hp, wr_ref[...], preferred_element_type=jnp.float32) + br_ref[...]


def _mid(acc1, bias1, gamma, beta, prelu_a, W_l2, b_l2, W_r2, b_r2):
    WD = 96
    g = NP_ // _MBLK
    h1, sums = pl.pallas_call(
        _mid1_body,
        grid=(g,),
        in_specs=[
            pl.BlockSpec((NC, _MBLK, WD), lambda i: (0, i, 0)),
            pl.BlockSpec((1, HD), lambda i: (0, 0)),
        ],
        out_specs=[
            pl.BlockSpec((_MBLK, HD), lambda i: (i, 0)),
            pl.BlockSpec((2, HD), lambda i: (0, 0)),
        ],
        out_shape=[
            jax.ShapeDtypeStruct((NP_, HD), jnp.float32),
            jax.ShapeDtypeStruct((2, HD), jnp.float32),
        ],
    )(acc1, bias1.reshape(1, HD))
    return pl.pallas_call(
        _mid2_body,
        grid=(g,),
        in_specs=[
            pl.BlockSpec((_MBLK, HD), lambda i: (i, 0)),
            pl.BlockSpec((2, HD), lambda i: (0, 0)),
            pl.BlockSpec((1, HD), lambda i: (0, 0)),
            pl.BlockSpec((1, HD), lambda i: (0, 0)),
            pl.BlockSpec((1, 1), lambda i: (0, 0)),
            pl.BlockSpec((HD, DIM_OUT), lambda i: (0, 0)),
            pl.BlockSpec((1, DIM_OUT), lambda i: (0, 0)),
            pl.BlockSpec((HD, DIM_OUT), lambda i: (0, 0)),
            pl.BlockSpec((1, DIM_OUT), lambda i: (0, 0)),
        ],
        out_specs=[
            pl.BlockSpec((_MBLK, DIM_OUT), lambda i: (i, 0)),
            pl.BlockSpec((_MBLK, DIM_OUT), lambda i: (i, 0)),
        ],
        out_shape=[
            jax.ShapeDtypeStruct((NP_, DIM_OUT), jnp.float32),
            jax.ShapeDtypeStruct((NP_, DIM_OUT), jnp.float32),
        ],
    )(h1, sums, gamma.reshape(1, HD), beta.reshape(1, HD),
      prelu_a.reshape(1, 1), W_l2, b_l2.reshape(1, DIM_OUT), W_r2,
      b_r2.reshape(1, DIM_OUT))


def _l2_score(XL2, XR2, att2, src3, dst3, nch):
    mesh = plsc.VectorSubcoreMesh(core_axis_name="c", subcore_axis_name="s")

    @functools.partial(
        pl.kernel,
        out_type=[
            jax.ShapeDtypeStruct((NW, nch, C), jnp.float32),
            jax.ShapeDtypeStruct((NW, 16), jnp.float32),
        ],
        mesh=mesh,
        compiler_params=pltpu.CompilerParams(needs_layout_passes=False, use_tc_tiling_on_sc=False),
        scratch_types=[
            pltpu.VMEM((nch, C), jnp.int32),
            pltpu.VMEM((nch, C), jnp.int32),
            pltpu.VMEM((C, DIM_OUT), jnp.float32),
            pltpu.VMEM((C, DIM_OUT), jnp.float32),
            pltpu.VMEM((C, DIM_OUT), jnp.float32),
            pltpu.VMEM((C, DIM_OUT), jnp.float32),
            pltpu.VMEM((C,), jnp.float32),
            pltpu.VMEM((C,), jnp.float32),
            pltpu.VMEM((4, 16), jnp.float32),
            pltpu.VMEM((16,), jnp.float32),
            pltpu.SemaphoreType.DMA,
            pltpu.SemaphoreType.DMA,
            pltpu.SemaphoreType.DMA,
            pltpu.SemaphoreType.DMA,
        ],
    )
    def k(xl_hbm, xr_hbm, att_hbm, src_hbm, dst_hbm, e_hbm, tmax_hbm,
          isrc, idst, xl0, xl1, xr0, xr1, e0, e1, att_v, tm_v,
          sl0, sl1, sr0, sr1):
        cid = lax.axis_index("c")
        sid = lax.axis_index("s")
        wid = sid * NC + cid
        pltpu.sync_copy(att_hbm, att_v)
        pltpu.sync_copy(src_hbm.at[wid], isrc)
        pltpu.sync_copy(dst_hbm.at[wid], idst)
        zv = jnp.zeros((16,), jnp.float32)
        lanes = lax.iota(jnp.int32, 16)
        atts = tuple(att_v[h] for h in range(4))
        bufs = ((xl0, xr0, e0, sl0, sr0), (xl1, xr1, e1, sl1, sr1))

        def issue(j, b):
            xl_v, xr_v, _, sl, sr = bufs[b]
            pltpu.async_copy(xl_hbm.at[isrc.at[j]], xl_v, sl)
            pltpu.async_copy(xr_hbm.at[idst.at[j]], xr_v, sr)

        issue(0, 0)
        issue(1, 1)

        def step(j, b, gmv):
            xl_v, xr_v, e_v, sl, sr = bufs[b]
            pltpu.make_async_copy(xl_hbm.at[isrc.at[j]], xl_v, sl).wait()
            pltpu.make_async_copy(xr_hbm.at[idst.at[j]], xr_v, sr).wait()

            @plsc.parallel_loop(0, C // 16, 1, unroll=2, carry=gmv)
            def gmv(kk, gm):
                evec = zv
                for ii in range(16):
                    i = kk * 16 + ii
                    sv = zv
                    for h in range(4):
                        a = xl_v[i, pl.ds(h * 16, 16)]
                        b_ = xr_v[i, pl.ds(h * 16, 16)]
                        m = a + b_
                        m = jnp.maximum(m, 0.2 * m)
                        sv = sv + m * atts[h]
                    evec = jnp.where(lanes == ii, jnp.sum(sv), evec)
                e_v[pl.ds(kk * 16, 16)] = evec
                return jnp.maximum(gm, evec)

            pltpu.sync_copy(e_v, e_hbm.at[wid, j])

            @pl.when(j + 2 < nch)
            def _():
                issue(j + 2, b)

            return gmv

        def body2(jj, gmv):
            gmv = step(2 * jj, 0, gmv)
            return step(2 * jj + 1, 1, gmv)

        gmv = lax.fori_loop(0, nch // 2, body2,
                            jnp.full((16,), -1e30, jnp.float32))
        tm_v[...] = gmv
        pltpu.sync_copy(tm_v, tmax_hbm.at[wid])

    return k(XL2, XR2, att2.reshape(4, 16), src3, dst3)


def _l2_agg(XL2, src3, dst3, e2, tmax2, nch):
    mesh = plsc.VectorSubcoreMesh(core_axis_name="c", subcore_axis_name="s")
    WD = DIM_OUT + 16

    @functools.partial(
        pl.kernel,
        out_type=jax.ShapeDtypeStruct((NC, NP_, WD), jnp.float32),
        mesh=mesh,
        compiler_params=pltpu.CompilerParams(needs_layout_passes=False, use_tc_tiling_on_sc=False),
        scratch_types=[
            pltpu.VMEM((nch, C), jnp.int32),
            pltpu.VMEM((nch, C), jnp.int32),
            pltpu.VMEM((C, DIM_OUT), jnp.float32),
            pltpu.VMEM((C, DIM_OUT), jnp.float32),
            pltpu.VMEM((C // 16, 16), jnp.float32),
            pltpu.VMEM((C // 16, 16), jnp.float32),
            pltpu.VMEM((C, WD), jnp.float32),
            pltpu.VMEM((C, WD), jnp.float32),
            pltpu.VMEM((NW, 16), jnp.float32),
            pltpu.VMEM_SHARED((NP_, WD), jnp.float32),
            pltpu.SemaphoreType.DMA,
            pltpu.SemaphoreType.DMA,
            pltpu.SemaphoreType.DMA,
            pltpu.SemaphoreType.DMA,
            pltpu.SemaphoreType.DMA,
            pltpu.SemaphoreType.DMA,
        ],
    )
    def k(xl_hbm, src_hbm, dst_hbm, e_hbm, tmax_hbm, acc_hbm,
          isrc, idst, xl0, xl1, e0, e1, w0, w1, tmax_v, acc_sh,
          sl0, sl1, se0, se1, ss0, ss1):
        cid = lax.axis_index("c")
        sid = lax.axis_index("s")
        wid = sid * NC + cid
        zv = jnp.zeros((16,), jnp.float32)

        def zrow(i, carry):
            for h in range(WD // 16):
                w0[i, pl.ds(h * 16, 16)] = zv
            return carry

        lax.fori_loop(0, C, zrow, 0)
        for t in range(ROWS_PER_TILE // C):
            pltpu.sync_copy(w0, acc_sh.at[pl.ds(sid * ROWS_PER_TILE + t * C, C)])
        plsc.subcore_barrier()

        pltpu.sync_copy(src_hbm.at[wid], isrc)
        pltpu.sync_copy(dst_hbm.at[wid], idst)
        pltpu.sync_copy(tmax_hbm, tmax_v)
        g = tmax_v[0]
        for w in range(1, NW):
            g = jnp.maximum(g, tmax_v[w])
        g = jnp.max(g)
        lanes = lax.iota(jnp.int32, 16)
        mask0 = jnp.where(lanes == 0, 1.0, 0.0)
        bufs = ((xl0, e0, w0, sl0, se0, ss0), (xl1, e1, w1, sl1, se1, ss1))

        def issue(j, b):
            xl_v, e_v, _, sl, se, _ = bufs[b]
            pltpu.async_copy(xl_hbm.at[isrc.at[j]], xl_v, sl)
            pltpu.async_copy(e_hbm.at[wid, j], e_v, se)

        issue(0, 0)
        issue(1, 1)

        def step(j, b):
            xl_v, e_v, w_v, sl, se, ss = bufs[b]
            pltpu.make_async_copy(xl_hbm.at[isrc.at[j]], xl_v, sl).wait()
            pltpu.make_async_copy(e_hbm.at[wid, j], e_v, se).wait()

            @pl.when(j >= 2)
            def _():
                pltpu.make_async_copy(w_v, acc_sh.at[idst.at[j - 2]], ss).wait()

            @plsc.parallel_loop(0, C // 16, 1, unroll=2)
            def _(kk):
                ev = e_v[kk, pl.ds(0, 16)]
                exv = jnp.exp(ev - g)
                for ii in range(16):
                    i = kk * 16 + ii
                    sc = exv[ii]
                    w_v[i, pl.ds(DIM_OUT, 16)] = mask0 * sc
                    for h in range(4):
                        w_v[i, pl.ds(h * 16, 16)] = xl_v[i, pl.ds(h * 16, 16)] * sc

            pltpu.async_copy(w_v, acc_sh.at[idst.at[j]], ss, add=True)

            @pl.when(j + 2 < nch)
            def _():
                issue(j + 2, b)

        def body2(jj, carry):
            step(2 * jj, 0)
            step(2 * jj + 1, 1)
            return carry

        lax.fori_loop(0, nch // 2, body2, 0)
        pltpu.make_async_copy(w0, acc_sh.at[idst.at[nch - 2]], ss0).wait()
        pltpu.make_async_copy(w1, acc_sh.at[idst.at[nch - 1]], ss1).wait()
        plsc.subcore_barrier()
        for t in range(ROWS_PER_TILE // C):
            base = sid * ROWS_PER_TILE + t * C
            pltpu.sync_copy(acc_sh.at[pl.ds(base, C)], acc_hbm.at[cid].at[pl.ds(base, C)])

    return k(XL2, src3, dst3, e2.reshape(NW, nch, C // 16, 16), tmax2)


def _post_body(acc_ref, b2_ref, out_ref, logp_ref):
    acc = acc_ref[0] + acc_ref[1]
    o = acc[:N, :DIM_OUT] / (acc[:N, DIM_OUT:DIM_OUT + 1] + 1e-16) + b2_ref[...]
    out_ref[...] = o
    mx = jnp.max(o, axis=1, keepdims=True)
    om = o - mx
    logp_ref[...] = om - jnp.log(jnp.sum(jnp.exp(om), axis=1, keepdims=True))


def _post(acc2, bias2):
    return pl.pallas_call(
        _post_body,
        out_shape=[
            jax.ShapeDtypeStruct((N, DIM_OUT), jnp.float32),
            jax.ShapeDtypeStruct((N, DIM_OUT), jnp.float32),
        ],
    )(acc2, bias2.reshape(1, DIM_OUT))


def kernel(x, edge_index, W_l1, b_l1, W_r1, b_r1, att1, bias1, gamma, beta,
           prelu_a, W_l2, b_l2, W_r2, b_r2, att2, bias2):
    n = x.shape[0]
    e_tot = edge_index.shape[1] + n
    nch = _ceil_div(e_tot, NW * C)
    e_pad = NW * nch * C
    loop = jnp.arange(n, dtype=jnp.int32)
    pad = jnp.full((e_pad - e_tot,), N, jnp.int32)
    src3 = jnp.concatenate([edge_index[0].astype(jnp.int32), loop, pad]).reshape(NW, nch, C)
    dst3 = jnp.concatenate([edge_index[1].astype(jnp.int32), loop, pad]).reshape(NW, nch, C)
    x_pad = jnp.pad(x, ((0, NP_ - n), (0, 0)))

    nch2 = e_pad // (NS * C)
    src_flat = src3.reshape(e_pad)
    src4 = jnp.stack([src_flat, src_flat + NP_]).reshape(NC, NS, nch2, C)
    dst4 = dst3.reshape(NS, nch2, C)

    XL1, XR1 = _project(x_pad, W_l1, b_l1, W_r1, b_r1)
    e1, tmax1 = _l1_score(XL1, XR1, att1, src3, dst3, nch)
    XLH = jnp.concatenate([XL1[:, :80], XL1[:, 80:]], axis=0)
    e4 = e1.reshape(NS, nch2, C, 16)
    acc1 = _l1_agg(XLH, src4, dst4, e4, tmax1, nch2)
    XL2, XR2 = _mid(acc1, bias1, gamma, beta, prelu_a, W_l2, b_l2, W_r2, b_r2)
    e2, tmax2 = _l2_score(XL2, XR2, att2, src3, dst3, nch)
    acc2 = _l2_agg(XL2, src3, dst3, e2, tmax2, nch)
    return _post(acc2, bias2)
